# Initial kernel scaffold; baseline (speedup 1.0000x reference)
import functools

import jax
import jax.numpy as jnp
from jax import lax
from jax.experimental import pallas as pl
from jax.experimental.pallas import tpu as pltpu
from jax.experimental.pallas import tpu_sc as plsc

B, N, D, K, M = 8, 576, 256, 1024, 8
NC, NS, L = 2, 16, 16
BPC = B // NC
WAUG = D + L
NT = N // NS
CH = 12
NCHUNK = NT // CH
CHM = CH * M
RT = K // NS
NVR = D // L


def _sc_mesh_kernel():
    mesh = plsc.VectorSubcoreMesh(
        core_axis_name="c", subcore_axis_name="s",
        num_cores=NC, num_subcores=NS)

    @functools.partial(
        pl.kernel,
        out_type=[
            jax.ShapeDtypeStruct((B * K, D), jnp.float32),
            jax.ShapeDtypeStruct((B, N, D), jnp.float32),
        ],
        mesh=mesh,
        scratch_types=[
            pltpu.VMEM_SHARED((BPC * K, WAUG), jnp.float32),
            pltpu.VMEM((RT, WAUG), jnp.float32),
            pltpu.VMEM((CH, D), jnp.float32),
            pltpu.VMEM((CHM, WAUG), jnp.float32),
            pltpu.VMEM((CHM,), jnp.int32),
            pltpu.VMEM((CHM,), jnp.float32),
            pltpu.VMEM((RT, D), jnp.float32),
            pltpu.VMEM((CHM, D), jnp.float32),
            pltpu.VMEM((CH, D), jnp.float32),
        ],
    )
    def sc_kernel(q_hbm, isc_hbm, igl_hbm, ws_hbm, wg_hbm,
                  proto_hbm, g_hbm,
                  table, tbuf, qbuf, sbuf, ibuf, wbuf, nbuf, gbuf, obuf):
        c = lax.axis_index("c")
        t = lax.axis_index("s")
        zero16 = jnp.zeros((L,), jnp.float32)
        lane0 = jnp.where(lax.iota(jnp.int32, L) == 0,
                          jnp.float32(1.0), jnp.float32(0.0))

        def zrow(r, carry):
            for v in range(WAUG // L):
                tbuf[r, pl.ds(v * L, L)] = zero16
            return carry
        lax.fori_loop(0, RT, zrow, 0)
        for j in range(BPC):
            pltpu.sync_copy(tbuf, table.at[pl.ds(t * BPC * RT + j * RT, RT)])
        plsc.subcore_barrier()

        def scatter_batch(b_local, carry):
            b = c * BPC + b_local

            def scatter_chunk(chunk, carry2):
                n0 = t * NT + chunk * CH
                pltpu.sync_copy(q_hbm.at[b, pl.ds(n0, CH), :], qbuf)
                pltpu.sync_copy(isc_hbm.at[b, pl.ds(n0 * M, CHM)], ibuf)
                pltpu.sync_copy(ws_hbm.at[b, pl.ds(n0 * M, CHM)], wbuf)

                def build(i, carry3):
                    for s in range(M):
                        r = i * M + s
                        wv = jnp.full((L,), wbuf[r], jnp.float32)
                        for v in range(NVR):
                            sbuf[r, pl.ds(v * L, L)] = (
                                qbuf[i, pl.ds(v * L, L)] * wv)
                        sbuf[r, pl.ds(D, L)] = wv * lane0
                    return carry3
                lax.fori_loop(0, CH, build, 0)
                pltpu.sync_copy(sbuf, table.at[ibuf], add=True)
                return carry2
            lax.fori_loop(0, NCHUNK, scatter_chunk, 0)
            return carry
        lax.fori_loop(0, BPC, scatter_batch, 0)
        plsc.subcore_barrier()

        def norm_batch(b_local, carry):
            b = c * BPC + b_local
            r0 = b_local * K + t * RT
            pltpu.sync_copy(table.at[pl.ds(r0, RT)], tbuf)

            def norm(i, carry2):
                dv = jnp.full((L,), tbuf[i, D], jnp.float32) + 1e-6
                inv = jnp.full((L,), 1.0, jnp.float32) / dv
                for v in range(NVR):
                    nbuf[i, pl.ds(v * L, L)] = tbuf[i, pl.ds(v * L, L)] * inv
                return carry2
            lax.fori_loop(0, RT, norm, 0)
            pltpu.sync_copy(nbuf, proto_hbm.at[pl.ds(b * K + t * RT, RT), :])
            return carry
        lax.fori_loop(0, BPC, norm_batch, 0)
        plsc.subcore_barrier()

        def gather_batch(b_local, carry):
            b = c * BPC + b_local

            def gather_chunk(chunk, carry2):
                n0 = t * NT + chunk * CH
                pltpu.sync_copy(igl_hbm.at[b, pl.ds(n0 * M, CHM)], ibuf)
                pltpu.sync_copy(wg_hbm.at[b, pl.ds(n0 * M, CHM)], wbuf)
                pltpu.sync_copy(proto_hbm.at[ibuf], gbuf)

                def wreduce(i, carry3):
                    accs = [jnp.zeros((L,), jnp.float32) for _ in range(NVR)]
                    for s in range(M):
                        r = i * M + s
                        wv = jnp.full((L,), wbuf[r], jnp.float32)
                        for v in range(NVR):
                            accs[v] = accs[v] + gbuf[r, pl.ds(v * L, L)] * wv
                    for v in range(NVR):
                        obuf[i, pl.ds(v * L, L)] = accs[v]
                    return carry3
                lax.fori_loop(0, CH, wreduce, 0)
                pltpu.sync_copy(obuf, g_hbm.at[b, pl.ds(n0, CH), :])
                return carry2
            lax.fori_loop(0, NCHUNK, gather_chunk, 0)
            return carry
        lax.fori_loop(0, BPC, gather_batch, 0)

    return sc_kernel


_SC_KERNEL = _sc_mesh_kernel()


def _tc_proto_body(x_ref, w_ref, b_ref, o_ref):
    o_ref[...] = jnp.dot(x_ref[...], w_ref[...],
                         preferred_element_type=jnp.float32) + b_ref[...]


def _tc_proto(x, wT, bvec):
    return pl.pallas_call(
        _tc_proto_body,
        grid=(8,),
        in_specs=[
            pl.BlockSpec((B * K // 8, D), lambda i: (i, 0)),
            pl.BlockSpec((D, D), lambda i: (0, 0)),
            pl.BlockSpec((1, D), lambda i: (0, 0)),
        ],
        out_specs=pl.BlockSpec((B * K // 8, D), lambda i: (i, 0)),
        out_shape=jax.ShapeDtypeStruct((B * K, D), jnp.float32),
    )(x, wT, bvec)


def _tc_update_body(q_ref, g_ref, sw_ref, wpp_ref, bpp_ref, wup_ref, bup_ref,
                    lng_ref, lnb_ref, o_ref):
    Qc = D // 4
    g = jnp.dot(g_ref[...], wpp_ref[...], preferred_element_type=jnp.float32)
    g = g + sw_ref[...][:, 0:1] * bpp_ref[...]
    qb = q_ref[...]
    pr, pi_, pj, pk = (qb[:, :Qc], qb[:, Qc:2 * Qc],
                       qb[:, 2 * Qc:3 * Qc], qb[:, 3 * Qc:])
    xr, xi, xj, xk = (g[:, :Qc], g[:, Qc:2 * Qc],
                      g[:, 2 * Qc:3 * Qc], g[:, 3 * Qc:])
    hr = pr * xr - pi_ * xi - pj * xj - pk * xk
    hi = pr * xi + pi_ * xr + pj * xk - pk * xj
    hj = pr * xj - pi_ * xk + pj * xr + pk * xi
    hk = pr * xk + pi_ * xj - pj * xi + pk * xr
    msg = jnp.concatenate([hr, hi, hj, hk], axis=1)
    out = jnp.dot(msg, wup_ref[...],
                  preferred_element_type=jnp.float32) + bup_ref[...]
    x = qb + out
    parts = []
    lng = lng_ref[...]
    lnb = lnb_ref[...]
    for ci in range(4):
        xc = x[:, ci * Qc:(ci + 1) * Qc]
        mu = jnp.mean(xc, axis=1, keepdims=True)
        xm = xc - mu
        var = jnp.mean(xm * xm, axis=1, keepdims=True)
        y = xm * lax.rsqrt(var + 1e-5)
        parts.append(y * lng[:, ci * Qc:(ci + 1) * Qc]
                     + lnb[:, ci * Qc:(ci + 1) * Qc])
    o_ref[...] = jnp.concatenate(parts, axis=1)


def _tc_update(qf, gf, swf, wppT, bpp, wupT, bup, lng, lnb):
    R = B * N // 9
    return pl.pallas_call(
        _tc_update_body,
        grid=(9,),
        in_specs=[
            pl.BlockSpec((R, D), lambda i: (i, 0)),
            pl.BlockSpec((R, D), lambda i: (i, 0)),
            pl.BlockSpec((R, 128), lambda i: (i, 0)),
            pl.BlockSpec((D, D), lambda i: (0, 0)),
            pl.BlockSpec((1, D), lambda i: (0, 0)),
            pl.BlockSpec((D, D), lambda i: (0, 0)),
            pl.BlockSpec((1, D), lambda i: (0, 0)),
            pl.BlockSpec((1, D), lambda i: (0, 0)),
            pl.BlockSpec((1, D), lambda i: (0, 0)),
        ],
        out_specs=pl.BlockSpec((R, D), lambda i: (i, 0)),
        out_shape=jax.ShapeDtypeStruct((B * N, D), jnp.float32),
    )(qf, gf, swf, wppT, bpp, wupT, bup, lng, lnb)


def _quat_weight(r, i, j, k):
    return jnp.concatenate([
        jnp.concatenate([r, -i, -j, -k], 1),
        jnp.concatenate([i, r, -k, j], 1),
        jnp.concatenate([j, k, r, -i], 1),
        jnp.concatenate([k, -j, i, r], 1)], 0)


def kernel(q, assign_idx, assign_w, contribute_mask,
           pp_r, pp_i, pp_j, pp_k, pp_b,
           up_r, up_i, up_j, up_k, up_b,
           ln_gr, ln_br, ln_gi, ln_bi, ln_gj, ln_bj, ln_gk, ln_bk):
    agg_w = assign_w * contribute_mask[..., None]
    idx = assign_idx.astype(jnp.int32)
    core_off = (jnp.arange(B, dtype=jnp.int32) % BPC) * K
    glob_off = jnp.arange(B, dtype=jnp.int32) * K
    idx_sc = (idx + core_off[:, None, None]).reshape(B, N * M)
    idx_gl = (idx + glob_off[:, None, None]).reshape(B, N * M)
    ws = agg_w.reshape(B, N * M)
    wg = assign_w.reshape(B, N * M)

    proto_norm, g_raw = _SC_KERNEL(q, idx_sc, idx_gl, ws, wg)

    wppT = _quat_weight(pp_r, pp_i, pp_j, pp_k).T
    wupT = _quat_weight(up_r, up_i, up_j, up_k).T
    proto_out = _tc_proto(proto_norm, wppT, pp_b.reshape(1, D))

    sw = jnp.broadcast_to(assign_w.sum(-1).reshape(B * N, 1), (B * N, 128))
    lng = jnp.concatenate([ln_gr, ln_gi, ln_gj, ln_gk]).reshape(1, D)
    lnb = jnp.concatenate([ln_br, ln_bi, ln_bj, ln_bk]).reshape(1, D)
    qn = _tc_update(q.reshape(B * N, D), g_raw.reshape(B * N, D), sw,
                    wppT, pp_b.reshape(1, D), wupT, up_b.reshape(1, D),
                    lng, lnb)
    return qn.reshape(B, N, D), proto_out.reshape(B, K, D)


# speedup vs baseline: 3.8384x; 3.8384x over previous
import functools

import jax
import jax.numpy as jnp
from jax import lax
from jax.experimental import pallas as pl
from jax.experimental.pallas import tpu as pltpu
from jax.experimental.pallas import tpu_sc as plsc

B, N, D, K, M = 8, 576, 256, 1024, 8
NC, NS, L = 2, 16, 16
BPC = B // NC
HW = 128
NHALF = BPC * K * 2
NT = N // NS
CH = 6
NCHUNK = NT // CH
CHM = CH * M
NVR = D // L
ZR = 32
NM = N * M


def _sc_mesh():
    return plsc.VectorSubcoreMesh(core_axis_name="c", subcore_axis_name="s",
                                  num_cores=NC, num_subcores=NS)


def _make_sc_scatter():
    @functools.partial(
        pl.kernel,
        out_type=[jax.ShapeDtypeStruct((NC * NHALF, HW), jnp.float32)],
        mesh=_sc_mesh(),
        scratch_types=[
            pltpu.VMEM_SHARED((NHALF, HW), jnp.float32),
            pltpu.VMEM((ZR, HW), jnp.float32),
            pltpu.VMEM((CH * D,), jnp.float32),
            pltpu.VMEM((2 * CHM, HW), jnp.float32),
            pltpu.VMEM((2 * CHM,), jnp.int32),
            pltpu.VMEM((CHM + L,), jnp.float32),
        ],
    )
    def sc_scatter(q_hbm, isc_hbm, ws_hbm, praw_hbm,
                   table, zbuf, qbuf, sbuf, ibuf, wbuf):
        c = lax.axis_index("c")
        t = lax.axis_index("s")
        zero16 = jnp.zeros((L,), jnp.float32)

        def zrow(r, carry):
            for v in range(HW // L):
                zbuf[r, pl.ds(v * L, L)] = zero16
            return carry
        lax.fori_loop(0, ZR, zrow, 0)

        def ztab(j, carry):
            pltpu.sync_copy(zbuf,
                            table.at[pl.ds(t * (NHALF // NS) + j * ZR, ZR)])
            return carry
        lax.fori_loop(0, NHALF // NS // ZR, ztab, 0)
        plsc.subcore_barrier()

        def scatter_batch(b_local, carry):
            b = c * BPC + b_local

            def scatter_chunk(chunk, carry2):
                n0 = t * NT + chunk * CH
                pltpu.sync_copy(q_hbm.at[pl.ds((b * N + n0) * D, CH * D)],
                                qbuf)
                pltpu.sync_copy(isc_hbm.at[pl.ds(b * NM + n0 * M, CHM)],
                                ibuf.at[pl.ds(0, CHM)])
                pltpu.sync_copy(ws_hbm.at[pl.ds(b * NM + n0 * M, CHM)],
                                wbuf.at[pl.ds(0, CHM)])
                for j in range(CHM // L):
                    iv = ibuf[pl.ds(j * L, L)]
                    ibuf[pl.ds(CHM + j * L, L)] = iv + 1

                def build(i, carry3):
                    wrow = wbuf[pl.ds(i * M, L)]
                    for s in range(M):
                        r = i * M + s
                        wv = jnp.full((L,), wrow[s], jnp.float32)
                        for v in range(NVR):
                            dst_r = r if v < 8 else CHM + r
                            dst_c = (v % 8) * L
                            sbuf[dst_r, pl.ds(dst_c, L)] = (
                                qbuf[pl.ds(i * D + v * L, L)] * wv)
                    return carry3
                lax.fori_loop(0, CH, build, 0)
                pltpu.sync_copy(sbuf, table.at[ibuf], add=True)
                return carry2
            lax.fori_loop(0, NCHUNK, scatter_chunk, 0)
            return carry
        lax.fori_loop(0, BPC, scatter_batch, 0)
        plsc.subcore_barrier()

        def dump(j, carry):
            r0 = t * (NHALF // NS) + j * ZR
            pltpu.sync_copy(table.at[pl.ds(r0, ZR)], zbuf)
            pltpu.sync_copy(zbuf, praw_hbm.at[pl.ds(c * NHALF + r0, ZR), :])
            return carry
        lax.fori_loop(0, NHALF // NS // ZR, dump, 0)

    return sc_scatter


def _make_sc_gather():
    @functools.partial(
        pl.kernel,
        out_type=[jax.ShapeDtypeStruct((B * N * D,), jnp.float32)],
        mesh=_sc_mesh(),
        scratch_types=[
            pltpu.VMEM((2 * CHM, HW), jnp.float32),
            pltpu.VMEM((2 * CHM,), jnp.int32),
            pltpu.VMEM((CHM + L,), jnp.float32),
            pltpu.VMEM((CH * D,), jnp.float32),
        ],
    )
    def sc_gather(pn_hbm, isc_hbm, wg_hbm, g_hbm, gbuf, ibuf, wbuf, obuf):
        c = lax.axis_index("c")
        t = lax.axis_index("s")
        coff = c * NHALF

        def gather_batch(b_local, carry):
            b = c * BPC + b_local

            def gather_chunk(chunk, carry2):
                n0 = t * NT + chunk * CH
                pltpu.sync_copy(isc_hbm.at[pl.ds(b * NM + n0 * M, CHM)],
                                ibuf.at[pl.ds(0, CHM)])
                pltpu.sync_copy(wg_hbm.at[pl.ds(b * NM + n0 * M, CHM)],
                                wbuf.at[pl.ds(0, CHM)])
                for j in range(CHM // L):
                    iv = ibuf[pl.ds(j * L, L)] + coff
                    ibuf[pl.ds(j * L, L)] = iv
                    ibuf[pl.ds(CHM + j * L, L)] = iv + 1
                pltpu.sync_copy(pn_hbm.at[ibuf], gbuf)

                def wreduce(i, carry3):
                    wrow = wbuf[pl.ds(i * M, L)]
                    accs = [jnp.zeros((L,), jnp.float32) for _ in range(NVR)]
                    for s in range(M):
                        r = i * M + s
                        wv = jnp.full((L,), wrow[s], jnp.float32)
                        for v in range(NVR):
                            src_r = r if v < 8 else CHM + r
                            src_c = (v % 8) * L
                            accs[v] = accs[v] + gbuf[src_r,
                                                     pl.ds(src_c, L)] * wv
                    for v in range(NVR):
                        obuf[pl.ds(i * D + v * L, L)] = accs[v]
                    return carry3
                lax.fori_loop(0, CH, wreduce, 0)
                pltpu.sync_copy(obuf,
                                g_hbm.at[pl.ds((b * N + n0) * D, CH * D)])
                return carry2
            lax.fori_loop(0, NCHUNK, gather_chunk, 0)
            return carry
        lax.fori_loop(0, BPC, gather_batch, 0)

    return sc_gather


_SC_SCATTER = _make_sc_scatter()
_SC_GATHER = _make_sc_gather()

_KCH = 256


def _tc_wsum_body(idx_ref, w_ref, o_ref):
    j = pl.program_id(0)
    kv = lax.broadcasted_iota(jnp.int32, (1, _KCH), 1) + j * _KCH
    for b in range(B):
        idxcol = idx_ref[...][:, b:b + 1]
        wcol = w_ref[...][:, b:b + 1]
        eq = idxcol == kv
        acc = jnp.sum(jnp.where(eq, wcol, 0.0), axis=0)
        o_ref[b, :] = acc


def _tc_wsum(idxp, wp):
    return pl.pallas_call(
        _tc_wsum_body,
        grid=(K // _KCH,),
        in_specs=[
            pl.BlockSpec((NM, 128), lambda j: (0, 0)),
            pl.BlockSpec((NM, 128), lambda j: (0, 0)),
        ],
        out_specs=pl.BlockSpec((B, _KCH), lambda j: (0, j)),
        out_shape=jax.ShapeDtypeStruct((B, K), jnp.float32),
    )(idxp, wp)


def _tc_norm_body(x_ref, ws_ref, w_ref, b_ref, on_ref, oo_ref):
    pn = x_ref[...] / (ws_ref[...][:, 0:1] + 1e-6)
    on_ref[...] = pn
    oo_ref[...] = jnp.dot(pn, w_ref[...],
                          preferred_element_type=jnp.float32) + b_ref[...]


def _tc_norm(praw, wsumb, wppT, bpp):
    R = B * K // 8
    return pl.pallas_call(
        _tc_norm_body,
        grid=(8,),
        in_specs=[
            pl.BlockSpec((R, D), lambda i: (i, 0)),
            pl.BlockSpec((R, 128), lambda i: (i, 0)),
            pl.BlockSpec((D, D), lambda i: (0, 0)),
            pl.BlockSpec((1, D), lambda i: (0, 0)),
        ],
        out_specs=[
            pl.BlockSpec((R, D), lambda i: (i, 0)),
            pl.BlockSpec((R, D), lambda i: (i, 0)),
        ],
        out_shape=[
            jax.ShapeDtypeStruct((B * K, D), jnp.float32),
            jax.ShapeDtypeStruct((B * K, D), jnp.float32),
        ],
    )(praw, wsumb, wppT, bpp)


def _tc_update_body(q_ref, g_ref, sw_ref, wpp_ref, bpp_ref, wup_ref, bup_ref,
                    lng_ref, lnb_ref, o_ref):
    Qc = D // 4
    g = jnp.dot(g_ref[...], wpp_ref[...], preferred_element_type=jnp.float32)
    g = g + sw_ref[...][:, 0:1] * bpp_ref[...]
    qb = q_ref[...]
    pr, pi_, pj, pk = (qb[:, :Qc], qb[:, Qc:2 * Qc],
                       qb[:, 2 * Qc:3 * Qc], qb[:, 3 * Qc:])
    xr, xi, xj, xk = (g[:, :Qc], g[:, Qc:2 * Qc],
                      g[:, 2 * Qc:3 * Qc], g[:, 3 * Qc:])
    hr = pr * xr - pi_ * xi - pj * xj - pk * xk
    hi = pr * xi + pi_ * xr + pj * xk - pk * xj
    hj = pr * xj - pi_ * xk + pj * xr + pk * xi
    hk = pr * xk + pi_ * xj - pj * xi + pk * xr
    msg = jnp.concatenate([hr, hi, hj, hk], axis=1)
    out = jnp.dot(msg, wup_ref[...],
                  preferred_element_type=jnp.float32) + bup_ref[...]
    x = qb + out
    parts = []
    lng = lng_ref[...]
    lnb = lnb_ref[...]
    for ci in range(4):
        xc = x[:, ci * Qc:(ci + 1) * Qc]
        mu = jnp.mean(xc, axis=1, keepdims=True)
        xm = xc - mu
        var = jnp.mean(xm * xm, axis=1, keepdims=True)
        y = xm * lax.rsqrt(var + 1e-5)
        parts.append(y * lng[:, ci * Qc:(ci + 1) * Qc]
                     + lnb[:, ci * Qc:(ci + 1) * Qc])
    o_ref[...] = jnp.concatenate(parts, axis=1)


def _tc_update(qf, gf, swf, wppT, bpp, wupT, bup, lng, lnb):
    R = B * N // 9
    return pl.pallas_call(
        _tc_update_body,
        grid=(9,),
        in_specs=[
            pl.BlockSpec((R, D), lambda i: (i, 0)),
            pl.BlockSpec((R, D), lambda i: (i, 0)),
            pl.BlockSpec((R, 128), lambda i: (i, 0)),
            pl.BlockSpec((D, D), lambda i: (0, 0)),
            pl.BlockSpec((1, D), lambda i: (0, 0)),
            pl.BlockSpec((D, D), lambda i: (0, 0)),
            pl.BlockSpec((1, D), lambda i: (0, 0)),
            pl.BlockSpec((1, D), lambda i: (0, 0)),
            pl.BlockSpec((1, D), lambda i: (0, 0)),
        ],
        out_specs=pl.BlockSpec((R, D), lambda i: (i, 0)),
        out_shape=jax.ShapeDtypeStruct((B * N, D), jnp.float32),
    )(qf, gf, swf, wppT, bpp, wupT, bup, lng, lnb)


def _quat_weight(r, i, j, k):
    return jnp.concatenate([
        jnp.concatenate([r, -i, -j, -k], 1),
        jnp.concatenate([i, r, -k, j], 1),
        jnp.concatenate([j, k, r, -i], 1),
        jnp.concatenate([k, -j, i, r], 1)], 0)


def kernel(q, assign_idx, assign_w, contribute_mask,
           pp_r, pp_i, pp_j, pp_k, pp_b,
           up_r, up_i, up_j, up_k, up_b,
           ln_gr, ln_br, ln_gi, ln_bi, ln_gj, ln_bj, ln_gk, ln_bk):
    agg_w = assign_w * contribute_mask[..., None]
    idx = assign_idx.astype(jnp.int32)
    core_off = (jnp.arange(B, dtype=jnp.int32) % BPC) * K
    isc2 = (2 * (idx + core_off[:, None, None])).reshape(B * NM)
    ws = agg_w.reshape(B * NM)
    wg = assign_w.reshape(B * NM)

    praw2 = _SC_SCATTER(q.reshape(B * N * D), isc2, ws)[0]

    idxp = jnp.zeros((NM, 128), jnp.int32).at[:, :B].set(
        idx.reshape(B, NM).T)
    wp = jnp.zeros((NM, 128), jnp.float32).at[:, :B].set(
        agg_w.reshape(B, NM).T)
    wsum = _tc_wsum(idxp, wp)
    wsumb = jnp.broadcast_to(wsum.reshape(B * K, 1), (B * K, 128))

    wppT = _quat_weight(pp_r, pp_i, pp_j, pp_k).T
    wupT = _quat_weight(up_r, up_i, up_j, up_k).T
    proto_norm, proto_out = _tc_norm(praw2.reshape(B * K, D), wsumb,
                                     wppT, pp_b.reshape(1, D))

    g_raw = _SC_GATHER(proto_norm.reshape(NC * NHALF, HW), isc2, wg)[0]

    sw = jnp.broadcast_to(assign_w.sum(-1).reshape(B * N, 1), (B * N, 128))
    lng = jnp.concatenate([ln_gr, ln_gi, ln_gj, ln_gk]).reshape(1, D)
    lnb = jnp.concatenate([ln_br, ln_bi, ln_bj, ln_bk]).reshape(1, D)
    qn = _tc_update(q.reshape(B * N, D), g_raw.reshape(B * N, D), sw,
                    wppT, pp_b.reshape(1, D), wupT, up_b.reshape(1, D),
                    lng, lnb)
    return qn.reshape(B, N, D), proto_out.reshape(B, K, D)


# speedup vs baseline: 4.5620x; 1.1885x over previous
import functools

import jax
import jax.numpy as jnp
from jax import lax
from jax.experimental import pallas as pl
from jax.experimental.pallas import tpu as pltpu
from jax.experimental.pallas import tpu_sc as plsc

B, N, D, K, M = 8, 576, 256, 1024, 8
NC, NS, L = 2, 16, 16
BPC = B // NC
HW = 128
NHALF = BPC * K * 2
NT = N // NS
CH = 12
NCHUNK = NT // CH
CHM = CH * M
NVR = D // L
ZR = 64
NM = N * M


def _sc_mesh():
    return plsc.VectorSubcoreMesh(core_axis_name="c", subcore_axis_name="s",
                                  num_cores=NC, num_subcores=NS)


def _make_sc_scatter():
    @functools.partial(
        pl.kernel,
        out_type=[jax.ShapeDtypeStruct((NC * NHALF, HW), jnp.float32)],
        mesh=_sc_mesh(),
        scratch_types=[
            pltpu.VMEM_SHARED((NHALF, HW), jnp.float32),
            pltpu.VMEM((ZR, HW), jnp.float32),
            pltpu.VMEM((CH * D,), jnp.float32),
            pltpu.VMEM((CHM, HW), jnp.float32),
            pltpu.VMEM((CHM, HW), jnp.float32),
            pltpu.VMEM((CHM,), jnp.int32),
            pltpu.VMEM((CHM,), jnp.int32),
            pltpu.VMEM((CHM + L,), jnp.float32),
            pltpu.SemaphoreType.DMA,
        ],
    )
    def sc_scatter(q_hbm, isc_hbm, ws_hbm, praw_hbm,
                   table, zbuf, qbuf, sblo, sbhi, iblo, ibhi, wbuf, lsem):
        c = lax.axis_index("c")
        t = lax.axis_index("s")
        zero16 = jnp.zeros((L,), jnp.float32)

        def zrow(r, carry):
            for v in range(HW // L):
                zbuf[r, pl.ds(v * L, L)] = zero16
            return carry
        lax.fori_loop(0, ZR, zrow, 0)

        def ztab(j, carry):
            pltpu.sync_copy(zbuf,
                            table.at[pl.ds(t * (NHALF // NS) + j * ZR, ZR)])
            return carry
        lax.fori_loop(0, NHALF // NS // ZR, ztab, 0)
        plsc.subcore_barrier()

        def scatter_batch(b_local, carry):
            b = c * BPC + b_local

            def scatter_chunk(chunk, carry2):
                n0 = t * NT + chunk * CH
                d1 = pltpu.async_copy(
                    q_hbm.at[pl.ds((b * N + n0) * D, CH * D)], qbuf, lsem)
                d2 = pltpu.async_copy(
                    isc_hbm.at[pl.ds(b * NM + n0 * M, CHM)], iblo, lsem)
                d3 = pltpu.async_copy(
                    ws_hbm.at[pl.ds(b * NM + n0 * M, CHM)],
                    wbuf.at[pl.ds(0, CHM)], lsem)
                d1.wait()
                d2.wait()
                d3.wait()
                for j in range(CHM // L):
                    iv = iblo[pl.ds(j * L, L)]
                    ibhi[pl.ds(j * L, L)] = iv + 1

                def build(i, carry3):
                    wrow = wbuf[pl.ds(i * M, L)]
                    for s in range(M):
                        r = i * M + s
                        wv = jnp.full((L,), wrow[s], jnp.float32)
                        for v in range(NVR):
                            dst = sblo if v < 8 else sbhi
                            dst_c = (v % 8) * L
                            dst[r, pl.ds(dst_c, L)] = (
                                qbuf[pl.ds(i * D + v * L, L)] * wv)
                    return carry3
                lax.fori_loop(0, CH, build, 0)
                pltpu.sync_copy(sblo, table.at[iblo], add=True)
                pltpu.sync_copy(sbhi, table.at[ibhi], add=True)
                return carry2
            lax.fori_loop(0, NCHUNK, scatter_chunk, 0)
            return carry
        lax.fori_loop(0, BPC, scatter_batch, 0)
        plsc.subcore_barrier()

        def dump(j, carry):
            r0 = t * (NHALF // NS) + j * ZR
            pltpu.sync_copy(table.at[pl.ds(r0, ZR)], zbuf)
            pltpu.sync_copy(zbuf, praw_hbm.at[pl.ds(c * NHALF + r0, ZR), :])
            return carry
        lax.fori_loop(0, NHALF // NS // ZR, dump, 0)

    return sc_scatter


def _make_sc_gather():
    @functools.partial(
        pl.kernel,
        out_type=[jax.ShapeDtypeStruct((B * N * D,), jnp.float32)],
        mesh=_sc_mesh(),
        scratch_types=[
            pltpu.VMEM((CHM, HW), jnp.float32),
            pltpu.VMEM((CHM, HW), jnp.float32),
            pltpu.VMEM((CHM,), jnp.int32),
            pltpu.VMEM((CHM,), jnp.int32),
            pltpu.VMEM((CHM + L,), jnp.float32),
            pltpu.VMEM((CH * D,), jnp.float32),
            pltpu.SemaphoreType.DMA,
        ],
    )
    def sc_gather(pn_hbm, isc_hbm, wg_hbm, g_hbm,
                  gblo, gbhi, iblo, ibhi, wbuf, obuf, lsem):
        c = lax.axis_index("c")
        t = lax.axis_index("s")
        coff = c * NHALF

        def gather_batch(b_local, carry):
            b = c * BPC + b_local

            def gather_chunk(chunk, carry2):
                n0 = t * NT + chunk * CH
                d1 = pltpu.async_copy(
                    isc_hbm.at[pl.ds(b * NM + n0 * M, CHM)], iblo, lsem)
                d2 = pltpu.async_copy(
                    wg_hbm.at[pl.ds(b * NM + n0 * M, CHM)],
                    wbuf.at[pl.ds(0, CHM)], lsem)
                d1.wait()
                d2.wait()
                for j in range(CHM // L):
                    iv = iblo[pl.ds(j * L, L)] + coff
                    iblo[pl.ds(j * L, L)] = iv
                    ibhi[pl.ds(j * L, L)] = iv + 1
                g1 = pltpu.async_copy(pn_hbm.at[iblo], gblo, lsem)
                g2 = pltpu.async_copy(pn_hbm.at[ibhi], gbhi, lsem)
                g1.wait()
                g2.wait()

                def wreduce(i, carry3):
                    wrow = wbuf[pl.ds(i * M, L)]
                    accs = [jnp.zeros((L,), jnp.float32) for _ in range(NVR)]
                    for s in range(M):
                        r = i * M + s
                        wv = jnp.full((L,), wrow[s], jnp.float32)
                        for v in range(NVR):
                            src = gblo if v < 8 else gbhi
                            src_c = (v % 8) * L
                            accs[v] = accs[v] + src[r, pl.ds(src_c, L)] * wv
                    for v in range(NVR):
                        obuf[pl.ds(i * D + v * L, L)] = accs[v]
                    return carry3
                lax.fori_loop(0, CH, wreduce, 0)
                pltpu.sync_copy(obuf,
                                g_hbm.at[pl.ds((b * N + n0) * D, CH * D)])
                return carry2
            lax.fori_loop(0, NCHUNK, gather_chunk, 0)
            return carry
        lax.fori_loop(0, BPC, gather_batch, 0)

    return sc_gather


_SC_SCATTER = _make_sc_scatter()
_SC_GATHER = _make_sc_gather()

_KCH = 256


def _tc_wsum_body(idx_ref, w_ref, o_ref):
    j = pl.program_id(0)
    kv = lax.broadcasted_iota(jnp.int32, (1, _KCH), 1) + j * _KCH
    for b in range(B):
        idxcol = idx_ref[...][:, b:b + 1]
        wcol = w_ref[...][:, b:b + 1]
        eq = idxcol == kv
        acc = jnp.sum(jnp.where(eq, wcol, 0.0), axis=0)
        o_ref[b, :] = acc


def _tc_wsum(idxp, wp):
    return pl.pallas_call(
        _tc_wsum_body,
        grid=(K // _KCH,),
        in_specs=[
            pl.BlockSpec((NM, 128), lambda j: (0, 0)),
            pl.BlockSpec((NM, 128), lambda j: (0, 0)),
        ],
        out_specs=pl.BlockSpec((B, _KCH), lambda j: (0, j)),
        out_shape=jax.ShapeDtypeStruct((B, K), jnp.float32),
    )(idxp, wp)


def _tc_norm_body(x_ref, ws_ref, w_ref, b_ref, on_ref, oo_ref):
    pn = x_ref[...] / (ws_ref[...][:, 0:1] + 1e-6)
    on_ref[...] = pn
    oo_ref[...] = jnp.dot(pn, w_ref[...],
                          preferred_element_type=jnp.float32) + b_ref[...]


def _tc_norm(praw, wsumb, wppT, bpp):
    R = B * K // 8
    return pl.pallas_call(
        _tc_norm_body,
        grid=(8,),
        in_specs=[
            pl.BlockSpec((R, D), lambda i: (i, 0)),
            pl.BlockSpec((R, 128), lambda i: (i, 0)),
            pl.BlockSpec((D, D), lambda i: (0, 0)),
            pl.BlockSpec((1, D), lambda i: (0, 0)),
        ],
        out_specs=[
            pl.BlockSpec((R, D), lambda i: (i, 0)),
            pl.BlockSpec((R, D), lambda i: (i, 0)),
        ],
        out_shape=[
            jax.ShapeDtypeStruct((B * K, D), jnp.float32),
            jax.ShapeDtypeStruct((B * K, D), jnp.float32),
        ],
    )(praw, wsumb, wppT, bpp)


def _tc_update_body(q_ref, g_ref, sw_ref, wpp_ref, bpp_ref, wup_ref, bup_ref,
                    lng_ref, lnb_ref, o_ref):
    Qc = D // 4
    g = jnp.dot(g_ref[...], wpp_ref[...], preferred_element_type=jnp.float32)
    g = g + sw_ref[...][:, 0:1] * bpp_ref[...]
    qb = q_ref[...]
    pr, pi_, pj, pk = (qb[:, :Qc], qb[:, Qc:2 * Qc],
                       qb[:, 2 * Qc:3 * Qc], qb[:, 3 * Qc:])
    xr, xi, xj, xk = (g[:, :Qc], g[:, Qc:2 * Qc],
                      g[:, 2 * Qc:3 * Qc], g[:, 3 * Qc:])
    hr = pr * xr - pi_ * xi - pj * xj - pk * xk
    hi = pr * xi + pi_ * xr + pj * xk - pk * xj
    hj = pr * xj - pi_ * xk + pj * xr + pk * xi
    hk = pr * xk + pi_ * xj - pj * xi + pk * xr
    msg = jnp.concatenate([hr, hi, hj, hk], axis=1)
    out = jnp.dot(msg, wup_ref[...],
                  preferred_element_type=jnp.float32) + bup_ref[...]
    x = qb + out
    parts = []
    lng = lng_ref[...]
    lnb = lnb_ref[...]
    for ci in range(4):
        xc = x[:, ci * Qc:(ci + 1) * Qc]
        mu = jnp.mean(xc, axis=1, keepdims=True)
        xm = xc - mu
        var = jnp.mean(xm * xm, axis=1, keepdims=True)
        y = xm * lax.rsqrt(var + 1e-5)
        parts.append(y * lng[:, ci * Qc:(ci + 1) * Qc]
                     + lnb[:, ci * Qc:(ci + 1) * Qc])
    o_ref[...] = jnp.concatenate(parts, axis=1)


def _tc_update(qf, gf, swf, wppT, bpp, wupT, bup, lng, lnb):
    R = B * N // 9
    return pl.pallas_call(
        _tc_update_body,
        grid=(9,),
        in_specs=[
            pl.BlockSpec((R, D), lambda i: (i, 0)),
            pl.BlockSpec((R, D), lambda i: (i, 0)),
            pl.BlockSpec((R, 128), lambda i: (i, 0)),
            pl.BlockSpec((D, D), lambda i: (0, 0)),
            pl.BlockSpec((1, D), lambda i: (0, 0)),
            pl.BlockSpec((D, D), lambda i: (0, 0)),
            pl.BlockSpec((1, D), lambda i: (0, 0)),
            pl.BlockSpec((1, D), lambda i: (0, 0)),
            pl.BlockSpec((1, D), lambda i: (0, 0)),
        ],
        out_specs=pl.BlockSpec((R, D), lambda i: (i, 0)),
        out_shape=jax.ShapeDtypeStruct((B * N, D), jnp.float32),
    )(qf, gf, swf, wppT, bpp, wupT, bup, lng, lnb)


def _quat_weight(r, i, j, k):
    return jnp.concatenate([
        jnp.concatenate([r, -i, -j, -k], 1),
        jnp.concatenate([i, r, -k, j], 1),
        jnp.concatenate([j, k, r, -i], 1),
        jnp.concatenate([k, -j, i, r], 1)], 0)


def kernel(q, assign_idx, assign_w, contribute_mask,
           pp_r, pp_i, pp_j, pp_k, pp_b,
           up_r, up_i, up_j, up_k, up_b,
           ln_gr, ln_br, ln_gi, ln_bi, ln_gj, ln_bj, ln_gk, ln_bk):
    agg_w = assign_w * contribute_mask[..., None]
    idx = assign_idx.astype(jnp.int32)
    core_off = (jnp.arange(B, dtype=jnp.int32) % BPC) * K
    isc2 = (2 * (idx + core_off[:, None, None])).reshape(B * NM)
    ws = agg_w.reshape(B * NM)
    wg = assign_w.reshape(B * NM)

    praw2 = _SC_SCATTER(q.reshape(B * N * D), isc2, ws)[0]

    idxp = jnp.zeros((NM, 128), jnp.int32).at[:, :B].set(
        idx.reshape(B, NM).T)
    wp = jnp.zeros((NM, 128), jnp.float32).at[:, :B].set(
        agg_w.reshape(B, NM).T)
    wsum = _tc_wsum(idxp, wp)
    wsumb = jnp.broadcast_to(wsum.reshape(B * K, 1), (B * K, 128))

    wppT = _quat_weight(pp_r, pp_i, pp_j, pp_k).T
    wupT = _quat_weight(up_r, up_i, up_j, up_k).T
    proto_norm, proto_out = _tc_norm(praw2.reshape(B * K, D), wsumb,
                                     wppT, pp_b.reshape(1, D))

    g_raw = _SC_GATHER(proto_norm.reshape(NC * NHALF, HW), isc2, wg)[0]

    sw = jnp.broadcast_to(assign_w.sum(-1).reshape(B * N, 1), (B * N, 128))
    lng = jnp.concatenate([ln_gr, ln_gi, ln_gj, ln_gk]).reshape(1, D)
    lnb = jnp.concatenate([ln_br, ln_bi, ln_bj, ln_bk]).reshape(1, D)
    qn = _tc_update(q.reshape(B * N, D), g_raw.reshape(B * N, D), sw,
                    wppT, pp_b.reshape(1, D), wupT, up_b.reshape(1, D),
                    lng, lnb)
    return qn.reshape(B, N, D), proto_out.reshape(B, K, D)


# speedup vs baseline: 4.9379x; 1.0824x over previous
import functools

import jax
import jax.numpy as jnp
from jax import lax
from jax.experimental import pallas as pl
from jax.experimental.pallas import tpu as pltpu
from jax.experimental.pallas import tpu_sc as plsc

B, N, D, K, M = 8, 576, 256, 1024, 8
NC, NS, L = 2, 16, 16
BPC = B // NC
HW = 128
NHALF = BPC * K * 2
NT = N // NS
CH = 12
NCHUNK = NT // CH
CHM = CH * M
NVR = D // L
ZR = 64
NM = N * M


def _sc_mesh():
    return plsc.VectorSubcoreMesh(core_axis_name="c", subcore_axis_name="s",
                                  num_cores=NC, num_subcores=NS)


def _make_sc_scatter():
    @functools.partial(
        pl.kernel,
        out_type=[jax.ShapeDtypeStruct((B * K, D), jnp.float32)],
        mesh=_sc_mesh(),
        scratch_types=[
            pltpu.VMEM_SHARED((NHALF, HW), jnp.float32),
            pltpu.VMEM((ZR, HW), jnp.float32),
            pltpu.VMEM((CH * D,), jnp.float32),
            pltpu.VMEM((CHM, HW), jnp.float32),
            pltpu.VMEM((CHM, HW), jnp.float32),
            pltpu.VMEM((CHM,), jnp.int32),
            pltpu.VMEM((CHM,), jnp.int32),
            pltpu.VMEM((CHM + L,), jnp.float32),
            pltpu.SemaphoreType.DMA,
        ],
    )
    def sc_scatter(q_hbm, isc_hbm, ws_hbm, praw_hbm,
                   table, zbuf, qbuf, sblo, sbhi, iblo, ibhi, wbuf, lsem):
        c = lax.axis_index("c")
        t = lax.axis_index("s")
        zero16 = jnp.zeros((L,), jnp.float32)

        def zrow(r, carry):
            for v in range(HW // L):
                zbuf[r, pl.ds(v * L, L)] = zero16
            return carry
        lax.fori_loop(0, ZR, zrow, 0)

        def ztab(j, carry):
            pltpu.sync_copy(zbuf,
                            table.at[pl.ds(t * (NHALF // NS) + j * ZR, ZR)])
            return carry
        lax.fori_loop(0, NHALF // NS // ZR, ztab, 0)
        plsc.subcore_barrier()

        def scatter_batch(b_local, carry):
            b = c * BPC + b_local

            def scatter_chunk(chunk, carry2):
                n0 = t * NT + chunk * CH
                d1 = pltpu.async_copy(
                    q_hbm.at[pl.ds((b * N + n0) * D, CH * D)], qbuf, lsem)
                d2 = pltpu.async_copy(
                    isc_hbm.at[pl.ds(b * NM + n0 * M, CHM)], iblo, lsem)
                d3 = pltpu.async_copy(
                    ws_hbm.at[pl.ds(b * NM + n0 * M, CHM)],
                    wbuf.at[pl.ds(0, CHM)], lsem)
                d1.wait()
                d2.wait()
                d3.wait()
                for j in range(CHM // L):
                    iv = iblo[pl.ds(j * L, L)]
                    ibhi[pl.ds(j * L, L)] = iv + BPC * K

                def build(i, carry3):
                    wrow = wbuf[pl.ds(i * M, L)]
                    for s in range(M):
                        r = i * M + s
                        wv = jnp.full((L,), wrow[s], jnp.float32)
                        for v in range(NVR):
                            dst = sblo if v < 8 else sbhi
                            dst_c = (v % 8) * L
                            dst[r, pl.ds(dst_c, L)] = (
                                qbuf[pl.ds(i * D + v * L, L)] * wv)
                    return carry3
                lax.fori_loop(0, CH, build, 0)
                pltpu.sync_copy(sblo, table.at[iblo], add=True)
                pltpu.sync_copy(sbhi, table.at[ibhi], add=True)
                return carry2
            lax.fori_loop(0, NCHUNK, scatter_chunk, 0)
            return carry
        lax.fori_loop(0, BPC, scatter_batch, 0)
        plsc.subcore_barrier()

        def dump_lo(j, carry):
            r0 = t * (BPC * K // NS) + j * ZR
            pltpu.sync_copy(table.at[pl.ds(r0, ZR)], zbuf)
            pltpu.sync_copy(zbuf, praw_hbm.at[pl.ds(c * BPC * K + r0, ZR),
                                              pl.ds(0, HW)])
            return carry
        lax.fori_loop(0, BPC * K // NS // ZR, dump_lo, 0)

        def dump_hi(j, carry):
            r0 = t * (BPC * K // NS) + j * ZR
            pltpu.sync_copy(table.at[pl.ds(BPC * K + r0, ZR)], zbuf)
            pltpu.sync_copy(zbuf, praw_hbm.at[pl.ds(c * BPC * K + r0, ZR),
                                              pl.ds(HW, HW)])
            return carry
        lax.fori_loop(0, BPC * K // NS // ZR, dump_hi, 0)

    return sc_scatter


def _make_sc_gather():
    @functools.partial(
        pl.kernel,
        out_type=[jax.ShapeDtypeStruct((B * N * D,), jnp.float32)],
        mesh=_sc_mesh(),
        scratch_types=[
            pltpu.VMEM((CHM, D), jnp.float32),
            pltpu.VMEM((CHM,), jnp.int32),
            pltpu.VMEM((CHM + L,), jnp.float32),
            pltpu.VMEM((CH * D,), jnp.float32),
            pltpu.SemaphoreType.DMA,
        ],
    )
    def sc_gather(pn_hbm, isc_hbm, wg_hbm, g_hbm,
                  gbuf, ibuf, wbuf, obuf, lsem):
        c = lax.axis_index("c")
        t = lax.axis_index("s")
        coff = c * BPC * K

        def gather_batch(b_local, carry):
            b = c * BPC + b_local

            def gather_chunk(chunk, carry2):
                n0 = t * NT + chunk * CH
                d1 = pltpu.async_copy(
                    isc_hbm.at[pl.ds(b * NM + n0 * M, CHM)], ibuf, lsem)
                d2 = pltpu.async_copy(
                    wg_hbm.at[pl.ds(b * NM + n0 * M, CHM)],
                    wbuf.at[pl.ds(0, CHM)], lsem)
                d1.wait()
                d2.wait()
                for j in range(CHM // L):
                    ibuf[pl.ds(j * L, L)] = ibuf[pl.ds(j * L, L)] + coff
                pltpu.sync_copy(pn_hbm.at[ibuf], gbuf)

                def wreduce(i, carry3):
                    wrow = wbuf[pl.ds(i * M, L)]
                    accs = [jnp.zeros((L,), jnp.float32) for _ in range(NVR)]
                    for s in range(M):
                        r = i * M + s
                        wv = jnp.full((L,), wrow[s], jnp.float32)
                        for v in range(NVR):
                            accs[v] = accs[v] + gbuf[r, pl.ds(v * L, L)] * wv
                    for v in range(NVR):
                        obuf[pl.ds(i * D + v * L, L)] = accs[v]
                    return carry3
                lax.fori_loop(0, CH, wreduce, 0)
                pltpu.sync_copy(obuf,
                                g_hbm.at[pl.ds((b * N + n0) * D, CH * D)])
                return carry2
            lax.fori_loop(0, NCHUNK, gather_chunk, 0)
            return carry
        lax.fori_loop(0, BPC, gather_batch, 0)

    return sc_gather


_SC_SCATTER = _make_sc_scatter()
_SC_GATHER = _make_sc_gather()

_KCH = 256


def _tc_wsum_body(idx_ref, w_ref, o_ref):
    j = pl.program_id(0)
    kv = lax.broadcasted_iota(jnp.int32, (1, _KCH), 1) + j * _KCH
    for b in range(B):
        idxcol = idx_ref[...][:, b:b + 1]
        wcol = w_ref[...][:, b:b + 1]
        eq = idxcol == kv
        acc = jnp.sum(jnp.where(eq, wcol, 0.0), axis=0)
        o_ref[b, :] = acc


def _tc_wsum(idxp, wp):
    return pl.pallas_call(
        _tc_wsum_body,
        grid=(K // _KCH,),
        in_specs=[
            pl.BlockSpec((NM, 128), lambda j: (0, 0)),
            pl.BlockSpec((NM, 128), lambda j: (0, 0)),
        ],
        out_specs=pl.BlockSpec((B, _KCH), lambda j: (0, j)),
        out_shape=jax.ShapeDtypeStruct((B, K), jnp.float32),
    )(idxp, wp)


def _tc_norm_body(x_ref, ws_ref, w_ref, b_ref, on_ref, oo_ref):
    pn = x_ref[...] / (ws_ref[...][:, 0:1] + 1e-6)
    on_ref[...] = pn
    oo_ref[...] = jnp.dot(pn, w_ref[...],
                          preferred_element_type=jnp.float32) + b_ref[...]


def _tc_norm(praw, wsumb, wppT, bpp):
    R = B * K // 8
    return pl.pallas_call(
        _tc_norm_body,
        grid=(8,),
        in_specs=[
            pl.BlockSpec((R, D), lambda i: (i, 0)),
            pl.BlockSpec((R, 128), lambda i: (i, 0)),
            pl.BlockSpec((D, D), lambda i: (0, 0)),
            pl.BlockSpec((1, D), lambda i: (0, 0)),
        ],
        out_specs=[
            pl.BlockSpec((R, D), lambda i: (i, 0)),
            pl.BlockSpec((R, D), lambda i: (i, 0)),
        ],
        out_shape=[
            jax.ShapeDtypeStruct((B * K, D), jnp.float32),
            jax.ShapeDtypeStruct((B * K, D), jnp.float32),
        ],
    )(praw, wsumb, wppT, bpp)


def _tc_update_body(q_ref, g_ref, sw_ref, wpp_ref, bpp_ref, wup_ref, bup_ref,
                    lng_ref, lnb_ref, o_ref):
    Qc = D // 4
    g = jnp.dot(g_ref[...], wpp_ref[...], preferred_element_type=jnp.float32)
    g = g + sw_ref[...][:, 0:1] * bpp_ref[...]
    qb = q_ref[...]
    pr, pi_, pj, pk = (qb[:, :Qc], qb[:, Qc:2 * Qc],
                       qb[:, 2 * Qc:3 * Qc], qb[:, 3 * Qc:])
    xr, xi, xj, xk = (g[:, :Qc], g[:, Qc:2 * Qc],
                      g[:, 2 * Qc:3 * Qc], g[:, 3 * Qc:])
    hr = pr * xr - pi_ * xi - pj * xj - pk * xk
    hi = pr * xi + pi_ * xr + pj * xk - pk * xj
    hj = pr * xj - pi_ * xk + pj * xr + pk * xi
    hk = pr * xk + pi_ * xj - pj * xi + pk * xr
    msg = jnp.concatenate([hr, hi, hj, hk], axis=1)
    out = jnp.dot(msg, wup_ref[...],
                  preferred_element_type=jnp.float32) + bup_ref[...]
    x = qb + out
    parts = []
    lng = lng_ref[...]
    lnb = lnb_ref[...]
    for ci in range(4):
        xc = x[:, ci * Qc:(ci + 1) * Qc]
        mu = jnp.mean(xc, axis=1, keepdims=True)
        xm = xc - mu
        var = jnp.mean(xm * xm, axis=1, keepdims=True)
        y = xm * lax.rsqrt(var + 1e-5)
        parts.append(y * lng[:, ci * Qc:(ci + 1) * Qc]
                     + lnb[:, ci * Qc:(ci + 1) * Qc])
    o_ref[...] = jnp.concatenate(parts, axis=1)


def _tc_update(qf, gf, swf, wppT, bpp, wupT, bup, lng, lnb):
    R = B * N // 9
    return pl.pallas_call(
        _tc_update_body,
        grid=(9,),
        in_specs=[
            pl.BlockSpec((R, D), lambda i: (i, 0)),
            pl.BlockSpec((R, D), lambda i: (i, 0)),
            pl.BlockSpec((R, 128), lambda i: (i, 0)),
            pl.BlockSpec((D, D), lambda i: (0, 0)),
            pl.BlockSpec((1, D), lambda i: (0, 0)),
            pl.BlockSpec((D, D), lambda i: (0, 0)),
            pl.BlockSpec((1, D), lambda i: (0, 0)),
            pl.BlockSpec((1, D), lambda i: (0, 0)),
            pl.BlockSpec((1, D), lambda i: (0, 0)),
        ],
        out_specs=pl.BlockSpec((R, D), lambda i: (i, 0)),
        out_shape=jax.ShapeDtypeStruct((B * N, D), jnp.float32),
    )(qf, gf, swf, wppT, bpp, wupT, bup, lng, lnb)


def _quat_weight(r, i, j, k):
    return jnp.concatenate([
        jnp.concatenate([r, -i, -j, -k], 1),
        jnp.concatenate([i, r, -k, j], 1),
        jnp.concatenate([j, k, r, -i], 1),
        jnp.concatenate([k, -j, i, r], 1)], 0)


def kernel(q, assign_idx, assign_w, contribute_mask,
           pp_r, pp_i, pp_j, pp_k, pp_b,
           up_r, up_i, up_j, up_k, up_b,
           ln_gr, ln_br, ln_gi, ln_bi, ln_gj, ln_bj, ln_gk, ln_bk):
    agg_w = assign_w * contribute_mask[..., None]
    idx = assign_idx.astype(jnp.int32)
    core_off = (jnp.arange(B, dtype=jnp.int32) % BPC) * K
    isc = (idx + core_off[:, None, None]).reshape(B * NM)
    ws = agg_w.reshape(B * NM)
    wg = assign_w.reshape(B * NM)

    praw = _SC_SCATTER(q.reshape(B * N * D), isc, ws)[0]

    idxp = jnp.zeros((NM, 128), jnp.int32).at[:, :B].set(
        idx.reshape(B, NM).T)
    wp = jnp.zeros((NM, 128), jnp.float32).at[:, :B].set(
        agg_w.reshape(B, NM).T)
    wsum = _tc_wsum(idxp, wp)
    wsumb = jnp.broadcast_to(wsum.reshape(B * K, 1), (B * K, 128))

    wppT = _quat_weight(pp_r, pp_i, pp_j, pp_k).T
    wupT = _quat_weight(up_r, up_i, up_j, up_k).T
    proto_norm, proto_out = _tc_norm(praw, wsumb, wppT, pp_b.reshape(1, D))

    g_raw = _SC_GATHER(proto_norm, isc, wg)[0]

    sw = jnp.broadcast_to(assign_w.sum(-1).reshape(B * N, 1), (B * N, 128))
    lng = jnp.concatenate([ln_gr, ln_gi, ln_gj, ln_gk]).reshape(1, D)
    lnb = jnp.concatenate([ln_br, ln_bi, ln_bj, ln_bk]).reshape(1, D)
    qn = _tc_update(q.reshape(B * N, D), g_raw.reshape(B * N, D), sw,
                    wppT, pp_b.reshape(1, D), wupT, up_b.reshape(1, D),
                    lng, lnb)
    return qn.reshape(B, N, D), proto_out.reshape(B, K, D)


# speedup vs baseline: 4.9713x; 1.0068x over previous
import functools

import jax
import jax.numpy as jnp
from jax import lax
from jax.experimental import pallas as pl
from jax.experimental.pallas import tpu as pltpu
from jax.experimental.pallas import tpu_sc as plsc

B, N, D, K, M = 8, 576, 256, 1024, 8
NC, NS, L = 2, 16, 16
BPC = B // NC
HW = 128
NHALF = BPC * K * 2
NT = N // NS
CH = 12
NCHUNK = NT // CH
CHM = CH * M
NVR = D // L
ZR = 64
NM = N * M


def _sc_mesh():
    return plsc.VectorSubcoreMesh(core_axis_name="c", subcore_axis_name="s",
                                  num_cores=NC, num_subcores=NS)


def _make_sc_scatter():
    @functools.partial(
        pl.kernel,
        out_type=[jax.ShapeDtypeStruct((B * K, D), jnp.float32)],
        mesh=_sc_mesh(),
        scratch_types=[
            pltpu.VMEM_SHARED((NHALF, HW), jnp.float32),
            pltpu.VMEM((ZR, HW), jnp.float32),
            pltpu.VMEM((CH * D,), jnp.float32),
            pltpu.VMEM((CHM, HW), jnp.float32),
            pltpu.VMEM((CHM, HW), jnp.float32),
            pltpu.VMEM((CHM,), jnp.int32),
            pltpu.VMEM((CHM,), jnp.int32),
            pltpu.VMEM((CHM + L,), jnp.float32),
            pltpu.SemaphoreType.DMA,
        ],
    )
    def sc_scatter(q_hbm, isc_hbm, ws_hbm, praw_hbm,
                   table, zbuf, qbuf, sblo, sbhi, iblo, ibhi, wbuf, lsem):
        c = lax.axis_index("c")
        t = lax.axis_index("s")
        zero16 = jnp.zeros((L,), jnp.float32)

        def zrow(r, carry):
            for v in range(HW // L):
                zbuf[r, pl.ds(v * L, L)] = zero16
            return carry
        lax.fori_loop(0, ZR, zrow, 0)

        zds = []
        for j in range(NHALF // NS // ZR):
            zds.append(pltpu.async_copy(
                zbuf, table.at[pl.ds(t * (NHALF // NS) + j * ZR, ZR)], lsem))
        for dz in zds:
            dz.wait()
        plsc.subcore_barrier()

        def scatter_batch(b_local, carry):
            b = c * BPC + b_local

            def scatter_chunk(chunk, carry2):
                n0 = t * NT + chunk * CH
                d1 = pltpu.async_copy(
                    q_hbm.at[pl.ds((b * N + n0) * D, CH * D)], qbuf, lsem)
                d2 = pltpu.async_copy(
                    isc_hbm.at[pl.ds(b * NM + n0 * M, CHM)], iblo, lsem)
                d3 = pltpu.async_copy(
                    ws_hbm.at[pl.ds(b * NM + n0 * M, CHM)],
                    wbuf.at[pl.ds(0, CHM)], lsem)
                d1.wait()
                d2.wait()
                d3.wait()
                for j in range(CHM // L):
                    iv = iblo[pl.ds(j * L, L)]
                    ibhi[pl.ds(j * L, L)] = iv + BPC * K

                def build(i, carry3):
                    wrow = wbuf[pl.ds(i * M, L)]
                    for s in range(M):
                        r = i * M + s
                        wv = jnp.full((L,), wrow[s], jnp.float32)
                        for v in range(NVR):
                            dst = sblo if v < 8 else sbhi
                            dst_c = (v % 8) * L
                            dst[r, pl.ds(dst_c, L)] = (
                                qbuf[pl.ds(i * D + v * L, L)] * wv)
                    return carry3
                lax.fori_loop(0, CH, build, 0)
                pltpu.sync_copy(sblo, table.at[iblo], add=True)
                pltpu.sync_copy(sbhi, table.at[ibhi], add=True)
                return carry2
            lax.fori_loop(0, NCHUNK, scatter_chunk, 0)
            return carry
        lax.fori_loop(0, BPC, scatter_batch, 0)
        plsc.subcore_barrier()

        RPT = BPC * K // NS
        r0 = t * RPT
        d_lo = pltpu.async_copy(
            table.at[pl.ds(r0, RPT)],
            praw_hbm.at[pl.ds(c * BPC * K + r0, RPT), pl.ds(0, HW)], lsem)
        d_hi = pltpu.async_copy(
            table.at[pl.ds(BPC * K + r0, RPT)],
            praw_hbm.at[pl.ds(c * BPC * K + r0, RPT), pl.ds(HW, HW)], lsem)
        d_lo.wait()
        d_hi.wait()

    return sc_scatter


def _make_sc_gather():
    @functools.partial(
        pl.kernel,
        out_type=[jax.ShapeDtypeStruct((B * N * D,), jnp.float32)],
        mesh=_sc_mesh(),
        scratch_types=[
            pltpu.VMEM((CHM, D), jnp.float32),
            pltpu.VMEM((CHM,), jnp.int32),
            pltpu.VMEM((CHM + L,), jnp.float32),
            pltpu.VMEM((CH * D,), jnp.float32),
            pltpu.SemaphoreType.DMA,
        ],
    )
    def sc_gather(pn_hbm, isc_hbm, wg_hbm, g_hbm,
                  gbuf, ibuf, wbuf, obuf, lsem):
        c = lax.axis_index("c")
        t = lax.axis_index("s")
        coff = c * BPC * K

        def gather_batch(b_local, carry):
            b = c * BPC + b_local

            def gather_chunk(chunk, carry2):
                n0 = t * NT + chunk * CH
                d1 = pltpu.async_copy(
                    isc_hbm.at[pl.ds(b * NM + n0 * M, CHM)], ibuf, lsem)
                d2 = pltpu.async_copy(
                    wg_hbm.at[pl.ds(b * NM + n0 * M, CHM)],
                    wbuf.at[pl.ds(0, CHM)], lsem)
                d1.wait()
                d2.wait()
                for j in range(CHM // L):
                    ibuf[pl.ds(j * L, L)] = ibuf[pl.ds(j * L, L)] + coff
                pltpu.sync_copy(pn_hbm.at[ibuf], gbuf)

                def wreduce(i, carry3):
                    wrow = wbuf[pl.ds(i * M, L)]
                    accs = [jnp.zeros((L,), jnp.float32) for _ in range(NVR)]
                    for s in range(M):
                        r = i * M + s
                        wv = jnp.full((L,), wrow[s], jnp.float32)
                        for v in range(NVR):
                            accs[v] = accs[v] + gbuf[r, pl.ds(v * L, L)] * wv
                    for v in range(NVR):
                        obuf[pl.ds(i * D + v * L, L)] = accs[v]
                    return carry3
                lax.fori_loop(0, CH, wreduce, 0)
                pltpu.sync_copy(obuf,
                                g_hbm.at[pl.ds((b * N + n0) * D, CH * D)])
                return carry2
            lax.fori_loop(0, NCHUNK, gather_chunk, 0)
            return carry
        lax.fori_loop(0, BPC, gather_batch, 0)

    return sc_gather


_SC_SCATTER = _make_sc_scatter()
_SC_GATHER = _make_sc_gather()

_KCH = 256


def _tc_wsum_body(idx_ref, w_ref, o_ref):
    j = pl.program_id(0)
    kv = lax.broadcasted_iota(jnp.int32, (1, _KCH), 1) + j * _KCH
    for b in range(B):
        idxcol = idx_ref[...][:, b:b + 1]
        wcol = w_ref[...][:, b:b + 1]
        eq = idxcol == kv
        acc = jnp.sum(jnp.where(eq, wcol, 0.0), axis=0)
        o_ref[b, :] = acc


def _tc_wsum(idxp, wp):
    return pl.pallas_call(
        _tc_wsum_body,
        grid=(K // _KCH,),
        in_specs=[
            pl.BlockSpec((NM, 128), lambda j: (0, 0)),
            pl.BlockSpec((NM, 128), lambda j: (0, 0)),
        ],
        out_specs=pl.BlockSpec((B, _KCH), lambda j: (0, j)),
        out_shape=jax.ShapeDtypeStruct((B, K), jnp.float32),
    )(idxp, wp)


def _tc_norm_body(x_ref, ws_ref, w_ref, b_ref, on_ref, oo_ref):
    pn = x_ref[...] / (ws_ref[...][:, 0:1] + 1e-6)
    on_ref[...] = pn
    oo_ref[...] = jnp.dot(pn, w_ref[...],
                          preferred_element_type=jnp.float32) + b_ref[...]


def _tc_norm(praw, wsumb, wppT, bpp):
    R = B * K // 8
    return pl.pallas_call(
        _tc_norm_body,
        grid=(8,),
        in_specs=[
            pl.BlockSpec((R, D), lambda i: (i, 0)),
            pl.BlockSpec((R, 128), lambda i: (i, 0)),
            pl.BlockSpec((D, D), lambda i: (0, 0)),
            pl.BlockSpec((1, D), lambda i: (0, 0)),
        ],
        out_specs=[
            pl.BlockSpec((R, D), lambda i: (i, 0)),
            pl.BlockSpec((R, D), lambda i: (i, 0)),
        ],
        out_shape=[
            jax.ShapeDtypeStruct((B * K, D), jnp.float32),
            jax.ShapeDtypeStruct((B * K, D), jnp.float32),
        ],
    )(praw, wsumb, wppT, bpp)


def _tc_update_body(q_ref, g_ref, sw_ref, wpp_ref, bpp_ref, wup_ref, bup_ref,
                    lng_ref, lnb_ref, o_ref):
    Qc = D // 4
    g = jnp.dot(g_ref[...], wpp_ref[...], preferred_element_type=jnp.float32)
    g = g + sw_ref[...][:, 0:1] * bpp_ref[...]
    qb = q_ref[...]
    pr, pi_, pj, pk = (qb[:, :Qc], qb[:, Qc:2 * Qc],
                       qb[:, 2 * Qc:3 * Qc], qb[:, 3 * Qc:])
    xr, xi, xj, xk = (g[:, :Qc], g[:, Qc:2 * Qc],
                      g[:, 2 * Qc:3 * Qc], g[:, 3 * Qc:])
    hr = pr * xr - pi_ * xi - pj * xj - pk * xk
    hi = pr * xi + pi_ * xr + pj * xk - pk * xj
    hj = pr * xj - pi_ * xk + pj * xr + pk * xi
    hk = pr * xk + pi_ * xj - pj * xi + pk * xr
    msg = jnp.concatenate([hr, hi, hj, hk], axis=1)
    out = jnp.dot(msg, wup_ref[...],
                  preferred_element_type=jnp.float32) + bup_ref[...]
    x = qb + out
    parts = []
    lng = lng_ref[...]
    lnb = lnb_ref[...]
    for ci in range(4):
        xc = x[:, ci * Qc:(ci + 1) * Qc]
        mu = jnp.mean(xc, axis=1, keepdims=True)
        xm = xc - mu
        var = jnp.mean(xm * xm, axis=1, keepdims=True)
        y = xm * lax.rsqrt(var + 1e-5)
        parts.append(y * lng[:, ci * Qc:(ci + 1) * Qc]
                     + lnb[:, ci * Qc:(ci + 1) * Qc])
    o_ref[...] = jnp.concatenate(parts, axis=1)


def _tc_update(qf, gf, swf, wppT, bpp, wupT, bup, lng, lnb):
    R = B * N // 9
    return pl.pallas_call(
        _tc_update_body,
        grid=(9,),
        in_specs=[
            pl.BlockSpec((R, D), lambda i: (i, 0)),
            pl.BlockSpec((R, D), lambda i: (i, 0)),
            pl.BlockSpec((R, 128), lambda i: (i, 0)),
            pl.BlockSpec((D, D), lambda i: (0, 0)),
            pl.BlockSpec((1, D), lambda i: (0, 0)),
            pl.BlockSpec((D, D), lambda i: (0, 0)),
            pl.BlockSpec((1, D), lambda i: (0, 0)),
            pl.BlockSpec((1, D), lambda i: (0, 0)),
            pl.BlockSpec((1, D), lambda i: (0, 0)),
        ],
        out_specs=pl.BlockSpec((R, D), lambda i: (i, 0)),
        out_shape=jax.ShapeDtypeStruct((B * N, D), jnp.float32),
    )(qf, gf, swf, wppT, bpp, wupT, bup, lng, lnb)


def _quat_weight(r, i, j, k):
    return jnp.concatenate([
        jnp.concatenate([r, -i, -j, -k], 1),
        jnp.concatenate([i, r, -k, j], 1),
        jnp.concatenate([j, k, r, -i], 1),
        jnp.concatenate([k, -j, i, r], 1)], 0)


def kernel(q, assign_idx, assign_w, contribute_mask,
           pp_r, pp_i, pp_j, pp_k, pp_b,
           up_r, up_i, up_j, up_k, up_b,
           ln_gr, ln_br, ln_gi, ln_bi, ln_gj, ln_bj, ln_gk, ln_bk):
    agg_w = assign_w * contribute_mask[..., None]
    idx = assign_idx.astype(jnp.int32)
    core_off = (jnp.arange(B, dtype=jnp.int32) % BPC) * K
    isc = (idx + core_off[:, None, None]).reshape(B * NM)
    ws = agg_w.reshape(B * NM)
    wg = assign_w.reshape(B * NM)

    praw = _SC_SCATTER(q.reshape(B * N * D), isc, ws)[0]

    idxp = jnp.zeros((NM, 128), jnp.int32).at[:, :B].set(
        idx.reshape(B, NM).T)
    wp = jnp.zeros((NM, 128), jnp.float32).at[:, :B].set(
        agg_w.reshape(B, NM).T)
    wsum = _tc_wsum(idxp, wp)
    wsumb = jnp.broadcast_to(wsum.reshape(B * K, 1), (B * K, 128))

    wppT = _quat_weight(pp_r, pp_i, pp_j, pp_k).T
    wupT = _quat_weight(up_r, up_i, up_j, up_k).T
    proto_norm, proto_out = _tc_norm(praw, wsumb, wppT, pp_b.reshape(1, D))

    g_raw = _SC_GATHER(proto_norm, isc, wg)[0]

    sw = jnp.broadcast_to(assign_w.sum(-1).reshape(B * N, 1), (B * N, 128))
    lng = jnp.concatenate([ln_gr, ln_gi, ln_gj, ln_gk]).reshape(1, D)
    lnb = jnp.concatenate([ln_br, ln_bi, ln_bj, ln_bk]).reshape(1, D)
    qn = _tc_update(q.reshape(B * N, D), g_raw.reshape(B * N, D), sw,
                    wppT, pp_b.reshape(1, D), wupT, up_b.reshape(1, D),
                    lng, lnb)
    return qn.reshape(B, N, D), proto_out.reshape(B, K, D)


# speedup vs baseline: 5.5273x; 1.1118x over previous
import functools

import jax
import jax.numpy as jnp
from jax import lax
from jax.experimental import pallas as pl
from jax.experimental.pallas import tpu as pltpu
from jax.experimental.pallas import tpu_sc as plsc

B, N, D, K, M = 8, 576, 256, 1024, 8
NC, NS, L = 2, 16, 16
BPC = B // NC
HW = 128
NHALF = BPC * K * 2
NT = N // NS
CH = 12
NCHUNK = NT // CH
CHM = CH * M
NVR = D // L
ZR = 64
NM = N * M


def _sc_mesh():
    return plsc.VectorSubcoreMesh(core_axis_name="c", subcore_axis_name="s",
                                  num_cores=NC, num_subcores=NS)


def _make_sc_scatter():
    @functools.partial(
        pl.kernel,
        out_type=[jax.ShapeDtypeStruct((B * K, D), jnp.float32)],
        mesh=_sc_mesh(),
        scratch_types=[
            pltpu.VMEM_SHARED((NHALF, HW), jnp.float32),
            pltpu.VMEM((ZR, HW), jnp.float32),
            pltpu.VMEM((CH * D,), jnp.float32),
            pltpu.VMEM((CH * D,), jnp.float32),
            pltpu.VMEM((CHM, HW), jnp.float32),
            pltpu.VMEM((CHM, HW), jnp.float32),
            pltpu.VMEM((CHM,), jnp.int32),
            pltpu.VMEM((CHM,), jnp.int32),
            pltpu.VMEM((BPC * NT * M,), jnp.int32),
            pltpu.VMEM((BPC * NT * M + L,), jnp.float32),
            pltpu.SemaphoreType.DMA,
            pltpu.SemaphoreType.DMA,
            pltpu.SemaphoreType.DMA,
        ],
    )
    def sc_scatter(q_hbm, isc_hbm, ws_hbm, praw_hbm,
                   table, zbuf, qbuf0, qbuf1, sblo, sbhi, iblo, ibhi,
                   ibig, wbig, lsem, qsem0, qsem1):
        c = lax.axis_index("c")
        t = lax.axis_index("s")
        zero16 = jnp.zeros((L,), jnp.float32)

        def zrow(r, carry):
            for v in range(HW // L):
                zbuf[r, pl.ds(v * L, L)] = zero16
            return carry
        lax.fori_loop(0, ZR, zrow, 0)

        zds = []
        for j in range(NHALF // NS // ZR):
            zds.append(pltpu.async_copy(
                zbuf, table.at[pl.ds(t * (NHALF // NS) + j * ZR, ZR)], lsem))
        for dz in zds:
            dz.wait()
        plsc.subcore_barrier()

        pds = []
        for bl in range(BPC):
            b0 = c * BPC + bl
            pds.append(pltpu.async_copy(
                isc_hbm.at[pl.ds(b0 * NM + t * NT * M, NT * M)],
                ibig.at[pl.ds(bl * NT * M, NT * M)], lsem))
            pds.append(pltpu.async_copy(
                ws_hbm.at[pl.ds(b0 * NM + t * NT * M, NT * M)],
                wbig.at[pl.ds(bl * NT * M, NT * M)], lsem))
        for dp in pds:
            dp.wait()

        qbufs = (qbuf0, qbuf1)
        qsems = (qsem0, qsem1)
        NCT = BPC * NCHUNK

        def q_src(j):
            b_local = j // NCHUNK
            chunk = j - b_local * NCHUNK
            b = c * BPC + b_local
            n0 = t * NT + chunk * CH
            return q_hbm.at[pl.ds((b * N + n0) * D, CH * D)]

        pltpu.async_copy(q_src(0), qbufs[0], qsems[0])

        def step(j, p):
            @pl.when(j < NCT - 1)
            def _():
                pltpu.async_copy(q_src(j + 1), qbufs[1 - p], qsems[1 - p])
            pltpu.make_async_copy(q_src(j), qbufs[p], qsems[p]).wait()
            b_local = j // NCHUNK
            chunk = j - b_local * NCHUNK
            base = b_local * NT * M + chunk * CHM
            for jj in range(CHM // L):
                iv = ibig[pl.ds(base + jj * L, L)]
                iblo[pl.ds(jj * L, L)] = iv
                ibhi[pl.ds(jj * L, L)] = iv + BPC * K
            qb = qbufs[p]

            def build(i, carry3):
                wrow = wbig[pl.ds(base + i * M, L)]
                for s in range(M):
                    r = i * M + s
                    wv = jnp.full((L,), wrow[s], jnp.float32)
                    for v in range(NVR):
                        dst = sblo if v < 8 else sbhi
                        dst_c = (v % 8) * L
                        dst[r, pl.ds(dst_c, L)] = (
                            qb[pl.ds(i * D + v * L, L)] * wv)
                return carry3
            lax.fori_loop(0, CH, build, 0)
            pltpu.sync_copy(sblo, table.at[iblo], add=True)
            pltpu.sync_copy(sbhi, table.at[ibhi], add=True)

        def super_step(g, carry):
            step(2 * g, 0)
            step(2 * g + 1, 1)
            return carry
        lax.fori_loop(0, NCT // 2, super_step, 0)
        plsc.subcore_barrier()

        RPT = BPC * K // NS
        r0 = t * RPT
        d_lo = pltpu.async_copy(
            table.at[pl.ds(r0, RPT)],
            praw_hbm.at[pl.ds(c * BPC * K + r0, RPT), pl.ds(0, HW)], lsem)
        d_hi = pltpu.async_copy(
            table.at[pl.ds(BPC * K + r0, RPT)],
            praw_hbm.at[pl.ds(c * BPC * K + r0, RPT), pl.ds(HW, HW)], lsem)
        d_lo.wait()
        d_hi.wait()

    return sc_scatter


def _make_sc_gather():
    @functools.partial(
        pl.kernel,
        out_type=[jax.ShapeDtypeStruct((B * N * D,), jnp.float32)],
        mesh=_sc_mesh(),
        scratch_types=[
            pltpu.VMEM((CHM, D), jnp.float32),
            pltpu.VMEM((CHM, D), jnp.float32),
            pltpu.VMEM((CHM,), jnp.int32),
            pltpu.VMEM((CHM,), jnp.int32),
            pltpu.VMEM((BPC * NT * M,), jnp.int32),
            pltpu.VMEM((BPC * NT * M + L,), jnp.float32),
            pltpu.VMEM((CH * D,), jnp.float32),
            pltpu.SemaphoreType.DMA,
            pltpu.SemaphoreType.DMA,
            pltpu.SemaphoreType.DMA,
        ],
    )
    def sc_gather(pn_hbm, isc_hbm, wg_hbm, g_hbm,
                  gbuf0, gbuf1, ibuf0, ibuf1, ibig, wbig, obuf,
                  lsem, gsem0, gsem1):
        c = lax.axis_index("c")
        t = lax.axis_index("s")
        coff = c * BPC * K

        pds = []
        for bl in range(BPC):
            b0 = c * BPC + bl
            pds.append(pltpu.async_copy(
                isc_hbm.at[pl.ds(b0 * NM + t * NT * M, NT * M)],
                ibig.at[pl.ds(bl * NT * M, NT * M)], lsem))
            pds.append(pltpu.async_copy(
                wg_hbm.at[pl.ds(b0 * NM + t * NT * M, NT * M)],
                wbig.at[pl.ds(bl * NT * M, NT * M)], lsem))
        for dp in pds:
            dp.wait()

        gbufs = (gbuf0, gbuf1)
        ibufs = (ibuf0, ibuf1)
        gsems = (gsem0, gsem1)
        NCT = BPC * NCHUNK

        def fill_idx(j, p):
            b_local = j // NCHUNK
            chunk = j - b_local * NCHUNK
            base = b_local * NT * M + chunk * CHM
            for jj in range(CHM // L):
                ibufs[p][pl.ds(jj * L, L)] = (
                    ibig[pl.ds(base + jj * L, L)] + coff)

        fill_idx(0, 0)
        pltpu.async_copy(pn_hbm.at[ibuf0], gbuf0, gsem0)

        def step(j, p):
            @pl.when(j < NCT - 1)
            def _():
                fill_idx(j + 1, 1 - p)
                pltpu.async_copy(pn_hbm.at[ibufs[1 - p]], gbufs[1 - p],
                                 gsems[1 - p])
            pltpu.make_async_copy(pn_hbm.at[ibufs[p]], gbufs[p],
                                  gsems[p]).wait()
            b_local = j // NCHUNK
            chunk = j - b_local * NCHUNK
            base = b_local * NT * M + chunk * CHM
            b = c * BPC + b_local
            n0 = t * NT + chunk * CH
            gb = gbufs[p]

            def wreduce(i, carry3):
                wrow = wbig[pl.ds(base + i * M, L)]
                accs = [jnp.zeros((L,), jnp.float32) for _ in range(NVR)]
                for s in range(M):
                    r = i * M + s
                    wv = jnp.full((L,), wrow[s], jnp.float32)
                    for v in range(NVR):
                        accs[v] = accs[v] + gb[r, pl.ds(v * L, L)] * wv
                for v in range(NVR):
                    obuf[pl.ds(i * D + v * L, L)] = accs[v]
                return carry3
            lax.fori_loop(0, CH, wreduce, 0)
            pltpu.sync_copy(obuf, g_hbm.at[pl.ds((b * N + n0) * D, CH * D)])

        def super_step(g, carry):
            step(2 * g, 0)
            step(2 * g + 1, 1)
            return carry
        lax.fori_loop(0, NCT // 2, super_step, 0)

    return sc_gather


_SC_SCATTER = _make_sc_scatter()
_SC_GATHER = _make_sc_gather()

_KCH = 256


def _tc_wsum_body(idx_ref, w_ref, o_ref):
    j = pl.program_id(0)
    kv = lax.broadcasted_iota(jnp.int32, (1, _KCH), 1) + j * _KCH
    for b in range(B):
        idxcol = idx_ref[...][:, b:b + 1]
        wcol = w_ref[...][:, b:b + 1]
        eq = idxcol == kv
        acc = jnp.sum(jnp.where(eq, wcol, 0.0), axis=0)
        o_ref[b, :] = acc


def _tc_wsum(idxp, wp):
    return pl.pallas_call(
        _tc_wsum_body,
        grid=(K // _KCH,),
        in_specs=[
            pl.BlockSpec((NM, 128), lambda j: (0, 0)),
            pl.BlockSpec((NM, 128), lambda j: (0, 0)),
        ],
        out_specs=pl.BlockSpec((B, _KCH), lambda j: (0, j)),
        out_shape=jax.ShapeDtypeStruct((B, K), jnp.float32),
    )(idxp, wp)


def _tc_norm_body(x_ref, ws_ref, w_ref, b_ref, on_ref, oo_ref):
    pn = x_ref[...] / (ws_ref[...][:, 0:1] + 1e-6)
    on_ref[...] = pn
    oo_ref[...] = jnp.dot(pn, w_ref[...],
                          preferred_element_type=jnp.float32) + b_ref[...]


def _tc_norm(praw, wsumb, wppT, bpp):
    R = B * K // 8
    return pl.pallas_call(
        _tc_norm_body,
        grid=(8,),
        in_specs=[
            pl.BlockSpec((R, D), lambda i: (i, 0)),
            pl.BlockSpec((R, 128), lambda i: (i, 0)),
            pl.BlockSpec((D, D), lambda i: (0, 0)),
            pl.BlockSpec((1, D), lambda i: (0, 0)),
        ],
        out_specs=[
            pl.BlockSpec((R, D), lambda i: (i, 0)),
            pl.BlockSpec((R, D), lambda i: (i, 0)),
        ],
        out_shape=[
            jax.ShapeDtypeStruct((B * K, D), jnp.float32),
            jax.ShapeDtypeStruct((B * K, D), jnp.float32),
        ],
    )(praw, wsumb, wppT, bpp)


def _tc_update_body(q_ref, g_ref, sw_ref, wpp_ref, bpp_ref, wup_ref, bup_ref,
                    lng_ref, lnb_ref, o_ref):
    Qc = D // 4
    g = jnp.dot(g_ref[...], wpp_ref[...], preferred_element_type=jnp.float32)
    g = g + sw_ref[...][:, 0:1] * bpp_ref[...]
    qb = q_ref[...]
    pr, pi_, pj, pk = (qb[:, :Qc], qb[:, Qc:2 * Qc],
                       qb[:, 2 * Qc:3 * Qc], qb[:, 3 * Qc:])
    xr, xi, xj, xk = (g[:, :Qc], g[:, Qc:2 * Qc],
                      g[:, 2 * Qc:3 * Qc], g[:, 3 * Qc:])
    hr = pr * xr - pi_ * xi - pj * xj - pk * xk
    hi = pr * xi + pi_ * xr + pj * xk - pk * xj
    hj = pr * xj - pi_ * xk + pj * xr + pk * xi
    hk = pr * xk + pi_ * xj - pj * xi + pk * xr
    msg = jnp.concatenate([hr, hi, hj, hk], axis=1)
    out = jnp.dot(msg, wup_ref[...],
                  preferred_element_type=jnp.float32) + bup_ref[...]
    x = qb + out
    parts = []
    lng = lng_ref[...]
    lnb = lnb_ref[...]
    for ci in range(4):
        xc = x[:, ci * Qc:(ci + 1) * Qc]
        mu = jnp.mean(xc, axis=1, keepdims=True)
        xm = xc - mu
        var = jnp.mean(xm * xm, axis=1, keepdims=True)
        y = xm * lax.rsqrt(var + 1e-5)
        parts.append(y * lng[:, ci * Qc:(ci + 1) * Qc]
                     + lnb[:, ci * Qc:(ci + 1) * Qc])
    o_ref[...] = jnp.concatenate(parts, axis=1)


def _tc_update(qf, gf, swf, wppT, bpp, wupT, bup, lng, lnb):
    R = B * N // 9
    return pl.pallas_call(
        _tc_update_body,
        grid=(9,),
        in_specs=[
            pl.BlockSpec((R, D), lambda i: (i, 0)),
            pl.BlockSpec((R, D), lambda i: (i, 0)),
            pl.BlockSpec((R, 128), lambda i: (i, 0)),
            pl.BlockSpec((D, D), lambda i: (0, 0)),
            pl.BlockSpec((1, D), lambda i: (0, 0)),
            pl.BlockSpec((D, D), lambda i: (0, 0)),
            pl.BlockSpec((1, D), lambda i: (0, 0)),
            pl.BlockSpec((1, D), lambda i: (0, 0)),
            pl.BlockSpec((1, D), lambda i: (0, 0)),
        ],
        out_specs=pl.BlockSpec((R, D), lambda i: (i, 0)),
        out_shape=jax.ShapeDtypeStruct((B * N, D), jnp.float32),
    )(qf, gf, swf, wppT, bpp, wupT, bup, lng, lnb)


def _quat_weight(r, i, j, k):
    return jnp.concatenate([
        jnp.concatenate([r, -i, -j, -k], 1),
        jnp.concatenate([i, r, -k, j], 1),
        jnp.concatenate([j, k, r, -i], 1),
        jnp.concatenate([k, -j, i, r], 1)], 0)


def kernel(q, assign_idx, assign_w, contribute_mask,
           pp_r, pp_i, pp_j, pp_k, pp_b,
           up_r, up_i, up_j, up_k, up_b,
           ln_gr, ln_br, ln_gi, ln_bi, ln_gj, ln_bj, ln_gk, ln_bk):
    agg_w = assign_w * contribute_mask[..., None]
    idx = assign_idx.astype(jnp.int32)
    core_off = (jnp.arange(B, dtype=jnp.int32) % BPC) * K
    isc = (idx + core_off[:, None, None]).reshape(B * NM)
    ws = agg_w.reshape(B * NM)
    wg = assign_w.reshape(B * NM)

    praw = _SC_SCATTER(q.reshape(B * N * D), isc, ws)[0]

    idxp = jnp.zeros((NM, 128), jnp.int32).at[:, :B].set(
        idx.reshape(B, NM).T)
    wp = jnp.zeros((NM, 128), jnp.float32).at[:, :B].set(
        agg_w.reshape(B, NM).T)
    wsum = _tc_wsum(idxp, wp)
    wsumb = jnp.broadcast_to(wsum.reshape(B * K, 1), (B * K, 128))

    wppT = _quat_weight(pp_r, pp_i, pp_j, pp_k).T
    wupT = _quat_weight(up_r, up_i, up_j, up_k).T
    proto_norm, proto_out = _tc_norm(praw, wsumb, wppT, pp_b.reshape(1, D))

    g_raw = _SC_GATHER(proto_norm, isc, wg)[0]

    sw = jnp.broadcast_to(assign_w.sum(-1).reshape(B * N, 1), (B * N, 128))
    lng = jnp.concatenate([ln_gr, ln_gi, ln_gj, ln_gk]).reshape(1, D)
    lnb = jnp.concatenate([ln_br, ln_bi, ln_bj, ln_bk]).reshape(1, D)
    qn = _tc_update(q.reshape(B * N, D), g_raw.reshape(B * N, D), sw,
                    wppT, pp_b.reshape(1, D), wupT, up_b.reshape(1, D),
                    lng, lnb)
    return qn.reshape(B, N, D), proto_out.reshape(B, K, D)


# speedup vs baseline: 6.6874x; 1.2099x over previous
import functools

import jax
import jax.numpy as jnp
from jax import lax
from jax.experimental import pallas as pl
from jax.experimental.pallas import tpu as pltpu
from jax.experimental.pallas import tpu_sc as plsc

B, N, D, K, M = 8, 576, 256, 1024, 8
NC, NS, L = 2, 16, 16
BPC = B // NC
HW = 128
NHALF = BPC * K * 2
NT = N // NS
CH = 12
NCHUNK = NT // CH
CHM = CH * M
NVR = D // L
ZR = 32
NM = N * M


def _sc_mesh():
    return plsc.VectorSubcoreMesh(core_axis_name="c", subcore_axis_name="s",
                                  num_cores=NC, num_subcores=NS)


def _make_sc_scatter():
    @functools.partial(
        pl.kernel,
        out_type=[jax.ShapeDtypeStruct((B * K, D), jnp.float32)],
        mesh=_sc_mesh(),
        scratch_types=[
            pltpu.VMEM_SHARED((NHALF, HW), jnp.float32),
            pltpu.VMEM((ZR, HW), jnp.float32),
            pltpu.VMEM((CH * D,), jnp.float32),
            pltpu.VMEM((CH * D,), jnp.float32),
            pltpu.VMEM((CHM, HW), jnp.float32),
            pltpu.VMEM((CHM, HW), jnp.float32),
            pltpu.VMEM((CHM, HW), jnp.float32),
            pltpu.VMEM((CHM, HW), jnp.float32),
            pltpu.VMEM((CHM,), jnp.int32),
            pltpu.VMEM((CHM,), jnp.int32),
            pltpu.VMEM((CHM,), jnp.int32),
            pltpu.VMEM((CHM,), jnp.int32),
            pltpu.VMEM((BPC * NT * M,), jnp.int32),
            pltpu.VMEM((BPC * NT * M + L,), jnp.float32),
            pltpu.SemaphoreType.DMA,
            pltpu.SemaphoreType.DMA,
            pltpu.SemaphoreType.DMA,
            pltpu.SemaphoreType.DMA,
            pltpu.SemaphoreType.DMA,
        ],
    )
    def sc_scatter(q_hbm, isc_hbm, ws_hbm, praw_hbm,
                   table, zbuf, qbuf0, qbuf1, sblo0, sbhi0, sblo1, sbhi1,
                   iblo0, ibhi0, iblo1, ibhi1,
                   ibig, wbig, lsem, qsem0, qsem1, ssem0, ssem1):
        c = lax.axis_index("c")
        t = lax.axis_index("s")
        zero16 = jnp.zeros((L,), jnp.float32)

        def zrow(r, carry):
            for v in range(HW // L):
                zbuf[r, pl.ds(v * L, L)] = zero16
            return carry
        lax.fori_loop(0, ZR, zrow, 0)

        zds = []
        for j in range(NHALF // NS // ZR):
            zds.append(pltpu.async_copy(
                zbuf, table.at[pl.ds(t * (NHALF // NS) + j * ZR, ZR)], lsem))
        for dz in zds:
            dz.wait()
        plsc.subcore_barrier()

        pds = []
        for bl in range(BPC):
            b0 = c * BPC + bl
            pds.append(pltpu.async_copy(
                isc_hbm.at[pl.ds(b0 * NM + t * NT * M, NT * M)],
                ibig.at[pl.ds(bl * NT * M, NT * M)], lsem))
            pds.append(pltpu.async_copy(
                ws_hbm.at[pl.ds(b0 * NM + t * NT * M, NT * M)],
                wbig.at[pl.ds(bl * NT * M, NT * M)], lsem))
        for dp in pds:
            dp.wait()

        qbufs = (qbuf0, qbuf1)
        qsems = (qsem0, qsem1)
        sblos = (sblo0, sblo1)
        sbhis = (sbhi0, sbhi1)
        iblos = (iblo0, iblo1)
        ibhis = (ibhi0, ibhi1)
        ssems = (ssem0, ssem1)
        NCT = BPC * NCHUNK

        def q_src(j):
            b_local = j // NCHUNK
            chunk = j - b_local * NCHUNK
            b = c * BPC + b_local
            n0 = t * NT + chunk * CH
            return q_hbm.at[pl.ds((b * N + n0) * D, CH * D)]

        pltpu.async_copy(q_src(0), qbufs[0], qsems[0])

        def wait_scatter(p):
            pltpu.make_async_copy(sblos[p], table.at[iblos[p]],
                                  ssems[p]).wait()
            pltpu.make_async_copy(sbhis[p], table.at[ibhis[p]],
                                  ssems[p]).wait()

        def step(j, p):
            @pl.when(j < NCT - 1)
            def _():
                pltpu.async_copy(q_src(j + 1), qbufs[1 - p], qsems[1 - p])
            pltpu.make_async_copy(q_src(j), qbufs[p], qsems[p]).wait()

            @pl.when(j >= 2)
            def _():
                wait_scatter(p)
            b_local = j // NCHUNK
            chunk = j - b_local * NCHUNK
            base = b_local * NT * M + chunk * CHM
            for jj in range(CHM // L):
                iv = ibig[pl.ds(base + jj * L, L)]
                iblos[p][pl.ds(jj * L, L)] = iv
                ibhis[p][pl.ds(jj * L, L)] = iv + BPC * K
            qb = qbufs[p]
            slo = sblos[p]
            shi = sbhis[p]

            def build(i, carry3):
                wrow = wbig[pl.ds(base + i * M, L)]
                qv = [qb[pl.ds(i * D + v * L, L)] for v in range(NVR)]
                for s in range(M):
                    r = i * M + s
                    wv = jnp.full((L,), wrow[s], jnp.float32)
                    for v in range(NVR):
                        dst = slo if v < 8 else shi
                        dst[r, pl.ds((v % 8) * L, L)] = qv[v] * wv
                return carry3
            lax.fori_loop(0, CH, build, 0)
            pltpu.async_copy(slo, table.at[iblos[p]], ssems[p], add=True)
            pltpu.async_copy(shi, table.at[ibhis[p]], ssems[p], add=True)

        def super_step(g, carry):
            step(2 * g, 0)
            step(2 * g + 1, 1)
            return carry
        lax.fori_loop(0, NCT // 2, super_step, 0)
        wait_scatter(0)
        wait_scatter(1)
        plsc.subcore_barrier()

        RPT = BPC * K // NS
        r0 = t * RPT
        d_lo = pltpu.async_copy(
            table.at[pl.ds(r0, RPT)],
            praw_hbm.at[pl.ds(c * BPC * K + r0, RPT), pl.ds(0, HW)], lsem)
        d_hi = pltpu.async_copy(
            table.at[pl.ds(BPC * K + r0, RPT)],
            praw_hbm.at[pl.ds(c * BPC * K + r0, RPT), pl.ds(HW, HW)], lsem)
        d_lo.wait()
        d_hi.wait()

    return sc_scatter


def _make_sc_gather():
    @functools.partial(
        pl.kernel,
        out_type=[jax.ShapeDtypeStruct((B * N * D,), jnp.float32)],
        mesh=_sc_mesh(),
        scratch_types=[
            pltpu.VMEM((CHM, D), jnp.float32),
            pltpu.VMEM((CHM, D), jnp.float32),
            pltpu.VMEM((CHM,), jnp.int32),
            pltpu.VMEM((CHM,), jnp.int32),
            pltpu.VMEM((BPC * NT * M,), jnp.int32),
            pltpu.VMEM((BPC * NT * M + L,), jnp.float32),
            pltpu.VMEM((CH * D,), jnp.float32),
            pltpu.SemaphoreType.DMA,
            pltpu.SemaphoreType.DMA,
            pltpu.SemaphoreType.DMA,
        ],
    )
    def sc_gather(pn_hbm, isc_hbm, wg_hbm, g_hbm,
                  gbuf0, gbuf1, ibuf0, ibuf1, ibig, wbig, obuf,
                  lsem, gsem0, gsem1):
        c = lax.axis_index("c")
        t = lax.axis_index("s")
        coff = c * BPC * K

        pds = []
        for bl in range(BPC):
            b0 = c * BPC + bl
            pds.append(pltpu.async_copy(
                isc_hbm.at[pl.ds(b0 * NM + t * NT * M, NT * M)],
                ibig.at[pl.ds(bl * NT * M, NT * M)], lsem))
            pds.append(pltpu.async_copy(
                wg_hbm.at[pl.ds(b0 * NM + t * NT * M, NT * M)],
                wbig.at[pl.ds(bl * NT * M, NT * M)], lsem))
        for dp in pds:
            dp.wait()

        gbufs = (gbuf0, gbuf1)
        ibufs = (ibuf0, ibuf1)
        gsems = (gsem0, gsem1)
        NCT = BPC * NCHUNK

        def fill_idx(j, p):
            b_local = j // NCHUNK
            chunk = j - b_local * NCHUNK
            base = b_local * NT * M + chunk * CHM
            for jj in range(CHM // L):
                ibufs[p][pl.ds(jj * L, L)] = (
                    ibig[pl.ds(base + jj * L, L)] + coff)

        fill_idx(0, 0)
        pltpu.async_copy(pn_hbm.at[ibuf0], gbuf0, gsem0)

        def step(j, p):
            @pl.when(j < NCT - 1)
            def _():
                fill_idx(j + 1, 1 - p)
                pltpu.async_copy(pn_hbm.at[ibufs[1 - p]], gbufs[1 - p],
                                 gsems[1 - p])
            pltpu.make_async_copy(pn_hbm.at[ibufs[p]], gbufs[p],
                                  gsems[p]).wait()
            b_local = j // NCHUNK
            chunk = j - b_local * NCHUNK
            base = b_local * NT * M + chunk * CHM
            b = c * BPC + b_local
            n0 = t * NT + chunk * CH
            gb = gbufs[p]

            def wreduce(i, carry3):
                wrow = wbig[pl.ds(base + i * M, L)]
                wvs = [jnp.full((L,), wrow[s], jnp.float32)
                       for s in range(M)]
                for v in range(NVR):
                    acc = gb[i * M, pl.ds(v * L, L)] * wvs[0]
                    for s in range(1, M):
                        acc = acc + gb[i * M + s, pl.ds(v * L, L)] * wvs[s]
                    obuf[pl.ds(i * D + v * L, L)] = acc
                return carry3
            lax.fori_loop(0, CH, wreduce, 0)
            pltpu.sync_copy(obuf, g_hbm.at[pl.ds((b * N + n0) * D, CH * D)])

        def super_step(g, carry):
            step(2 * g, 0)
            step(2 * g + 1, 1)
            return carry
        lax.fori_loop(0, NCT // 2, super_step, 0)

    return sc_gather


_SC_SCATTER = _make_sc_scatter()
_SC_GATHER = _make_sc_gather()

_KCH = 256


def _tc_wsum_body(idx_ref, w_ref, o_ref):
    j = pl.program_id(0)
    kv = lax.broadcasted_iota(jnp.int32, (1, _KCH), 1) + j * _KCH
    for b in range(B):
        idxcol = idx_ref[...][:, b:b + 1]
        wcol = w_ref[...][:, b:b + 1]
        eq = idxcol == kv
        acc = jnp.sum(jnp.where(eq, wcol, 0.0), axis=0)
        o_ref[b, :] = acc


def _tc_wsum(idxp, wp):
    return pl.pallas_call(
        _tc_wsum_body,
        grid=(K // _KCH,),
        in_specs=[
            pl.BlockSpec((NM, 128), lambda j: (0, 0)),
            pl.BlockSpec((NM, 128), lambda j: (0, 0)),
        ],
        out_specs=pl.BlockSpec((B, _KCH), lambda j: (0, j)),
        out_shape=jax.ShapeDtypeStruct((B, K), jnp.float32),
    )(idxp, wp)


def _tc_norm_body(x_ref, ws_ref, w_ref, b_ref, on_ref, oo_ref):
    pn = x_ref[...] / (ws_ref[...][:, 0:1] + 1e-6)
    on_ref[...] = pn
    oo_ref[...] = jnp.dot(pn, w_ref[...],
                          preferred_element_type=jnp.float32) + b_ref[...]


def _tc_norm(praw, wsumb, wppT, bpp):
    R = B * K // 8
    return pl.pallas_call(
        _tc_norm_body,
        grid=(8,),
        in_specs=[
            pl.BlockSpec((R, D), lambda i: (i, 0)),
            pl.BlockSpec((R, 128), lambda i: (i, 0)),
            pl.BlockSpec((D, D), lambda i: (0, 0)),
            pl.BlockSpec((1, D), lambda i: (0, 0)),
        ],
        out_specs=[
            pl.BlockSpec((R, D), lambda i: (i, 0)),
            pl.BlockSpec((R, D), lambda i: (i, 0)),
        ],
        out_shape=[
            jax.ShapeDtypeStruct((B * K, D), jnp.float32),
            jax.ShapeDtypeStruct((B * K, D), jnp.float32),
        ],
    )(praw, wsumb, wppT, bpp)


def _tc_update_body(q_ref, g_ref, sw_ref, wpp_ref, bpp_ref, wup_ref, bup_ref,
                    lng_ref, lnb_ref, o_ref):
    Qc = D // 4
    g = jnp.dot(g_ref[...], wpp_ref[...], preferred_element_type=jnp.float32)
    g = g + sw_ref[...][:, 0:1] * bpp_ref[...]
    qb = q_ref[...]
    pr, pi_, pj, pk = (qb[:, :Qc], qb[:, Qc:2 * Qc],
                       qb[:, 2 * Qc:3 * Qc], qb[:, 3 * Qc:])
    xr, xi, xj, xk = (g[:, :Qc], g[:, Qc:2 * Qc],
                      g[:, 2 * Qc:3 * Qc], g[:, 3 * Qc:])
    hr = pr * xr - pi_ * xi - pj * xj - pk * xk
    hi = pr * xi + pi_ * xr + pj * xk - pk * xj
    hj = pr * xj - pi_ * xk + pj * xr + pk * xi
    hk = pr * xk + pi_ * xj - pj * xi + pk * xr
    msg = jnp.concatenate([hr, hi, hj, hk], axis=1)
    out = jnp.dot(msg, wup_ref[...],
                  preferred_element_type=jnp.float32) + bup_ref[...]
    x = qb + out
    parts = []
    lng = lng_ref[...]
    lnb = lnb_ref[...]
    for ci in range(4):
        xc = x[:, ci * Qc:(ci + 1) * Qc]
        mu = jnp.mean(xc, axis=1, keepdims=True)
        xm = xc - mu
        var = jnp.mean(xm * xm, axis=1, keepdims=True)
        y = xm * lax.rsqrt(var + 1e-5)
        parts.append(y * lng[:, ci * Qc:(ci + 1) * Qc]
                     + lnb[:, ci * Qc:(ci + 1) * Qc])
    o_ref[...] = jnp.concatenate(parts, axis=1)


def _tc_update(qf, gf, swf, wppT, bpp, wupT, bup, lng, lnb):
    R = B * N // 9
    return pl.pallas_call(
        _tc_update_body,
        grid=(9,),
        in_specs=[
            pl.BlockSpec((R, D), lambda i: (i, 0)),
            pl.BlockSpec((R, D), lambda i: (i, 0)),
            pl.BlockSpec((R, 128), lambda i: (i, 0)),
            pl.BlockSpec((D, D), lambda i: (0, 0)),
            pl.BlockSpec((1, D), lambda i: (0, 0)),
            pl.BlockSpec((D, D), lambda i: (0, 0)),
            pl.BlockSpec((1, D), lambda i: (0, 0)),
            pl.BlockSpec((1, D), lambda i: (0, 0)),
            pl.BlockSpec((1, D), lambda i: (0, 0)),
        ],
        out_specs=pl.BlockSpec((R, D), lambda i: (i, 0)),
        out_shape=jax.ShapeDtypeStruct((B * N, D), jnp.float32),
    )(qf, gf, swf, wppT, bpp, wupT, bup, lng, lnb)


def _quat_weight(r, i, j, k):
    return jnp.concatenate([
        jnp.concatenate([r, -i, -j, -k], 1),
        jnp.concatenate([i, r, -k, j], 1),
        jnp.concatenate([j, k, r, -i], 1),
        jnp.concatenate([k, -j, i, r], 1)], 0)


def kernel(q, assign_idx, assign_w, contribute_mask,
           pp_r, pp_i, pp_j, pp_k, pp_b,
           up_r, up_i, up_j, up_k, up_b,
           ln_gr, ln_br, ln_gi, ln_bi, ln_gj, ln_bj, ln_gk, ln_bk):
    agg_w = assign_w * contribute_mask[..., None]
    idx = assign_idx.astype(jnp.int32)
    core_off = (jnp.arange(B, dtype=jnp.int32) % BPC) * K
    isc = (idx + core_off[:, None, None]).reshape(B * NM)
    ws = agg_w.reshape(B * NM)
    wg = assign_w.reshape(B * NM)

    praw = _SC_SCATTER(q.reshape(B * N * D), isc, ws)[0]

    idxp = jnp.zeros((NM, 128), jnp.int32).at[:, :B].set(
        idx.reshape(B, NM).T)
    wp = jnp.zeros((NM, 128), jnp.float32).at[:, :B].set(
        agg_w.reshape(B, NM).T)
    wsum = _tc_wsum(idxp, wp)
    wsumb = jnp.broadcast_to(wsum.reshape(B * K, 1), (B * K, 128))

    wppT = _quat_weight(pp_r, pp_i, pp_j, pp_k).T
    wupT = _quat_weight(up_r, up_i, up_j, up_k).T
    proto_norm, proto_out = _tc_norm(praw, wsumb, wppT, pp_b.reshape(1, D))

    g_raw = _SC_GATHER(proto_norm, isc, wg)[0]

    sw = jnp.broadcast_to(assign_w.sum(-1).reshape(B * N, 1), (B * N, 128))
    lng = jnp.concatenate([ln_gr, ln_gi, ln_gj, ln_gk]).reshape(1, D)
    lnb = jnp.concatenate([ln_br, ln_bi, ln_bj, ln_bk]).reshape(1, D)
    qn = _tc_update(q.reshape(B * N, D), g_raw.reshape(B * N, D), sw,
                    wppT, pp_b.reshape(1, D), wupT, up_b.reshape(1, D),
                    lng, lnb)
    return qn.reshape(B, N, D), proto_out.reshape(B, K, D)


# speedup vs baseline: 9.1612x; 1.3699x over previous
import functools

import jax
import jax.numpy as jnp
from jax import lax
from jax.experimental import pallas as pl
from jax.experimental.pallas import tpu as pltpu
from jax.experimental.pallas import tpu_sc as plsc

B, N, D, K, M = 8, 576, 256, 1024, 8
NC, NS, L = 2, 16, 16
BPC = B // NC
HW = 128
NHALF = BPC * K * 2
NT = N // NS
CH = 12
NCHUNK = NT // CH
CHM = CH * M
NVR = D // L
ZR = 32
NM = N * M


def _sc_mesh():
    return plsc.VectorSubcoreMesh(core_axis_name="c", subcore_axis_name="s",
                                  num_cores=NC, num_subcores=NS)


def _make_sc_scatter():
    @functools.partial(
        pl.kernel,
        out_type=[jax.ShapeDtypeStruct((B * K, D), jnp.float32)],
        mesh=_sc_mesh(),
        scratch_types=[
            pltpu.VMEM_SHARED((NHALF, HW), jnp.float32),
            pltpu.VMEM((ZR, HW), jnp.float32),
            pltpu.VMEM((CH * D,), jnp.float32),
            pltpu.VMEM((CH * D,), jnp.float32),
            pltpu.VMEM((CHM, HW), jnp.float32),
            pltpu.VMEM((CHM, HW), jnp.float32),
            pltpu.VMEM((CHM, HW), jnp.float32),
            pltpu.VMEM((CHM, HW), jnp.float32),
            pltpu.VMEM((CHM,), jnp.int32),
            pltpu.VMEM((CHM,), jnp.int32),
            pltpu.VMEM((CHM,), jnp.int32),
            pltpu.VMEM((CHM,), jnp.int32),
            pltpu.VMEM((BPC * NT * M,), jnp.int32),
            pltpu.VMEM((BPC * NT * M + L,), jnp.float32),
            pltpu.SemaphoreType.DMA,
            pltpu.SemaphoreType.DMA,
            pltpu.SemaphoreType.DMA,
            pltpu.SemaphoreType.DMA,
            pltpu.SemaphoreType.DMA,
        ],
    )
    def sc_scatter(q_hbm, isc_hbm, ws_hbm, praw_hbm,
                   table, zbuf, qbuf0, qbuf1, sblo0, sbhi0, sblo1, sbhi1,
                   iblo0, ibhi0, iblo1, ibhi1,
                   ibig, wbig, lsem, qsem0, qsem1, ssem0, ssem1):
        c = lax.axis_index("c")
        t = lax.axis_index("s")
        zero16 = jnp.zeros((L,), jnp.float32)

        def zrow(r, carry):
            for v in range(HW // L):
                zbuf[r, pl.ds(v * L, L)] = zero16
            return carry
        lax.fori_loop(0, ZR, zrow, 0)

        zds = []
        for j in range(NHALF // NS // ZR):
            zds.append(pltpu.async_copy(
                zbuf, table.at[pl.ds(t * (NHALF // NS) + j * ZR, ZR)], lsem))
        for dz in zds:
            dz.wait()
        plsc.subcore_barrier()

        pds = []
        for bl in range(BPC):
            b0 = c * BPC + bl
            pds.append(pltpu.async_copy(
                isc_hbm.at[pl.ds(b0 * NM + t * NT * M, NT * M)],
                ibig.at[pl.ds(bl * NT * M, NT * M)], lsem))
            pds.append(pltpu.async_copy(
                ws_hbm.at[pl.ds(b0 * NM + t * NT * M, NT * M)],
                wbig.at[pl.ds(bl * NT * M, NT * M)], lsem))
        for dp in pds:
            dp.wait()

        qbufs = (qbuf0, qbuf1)
        qsems = (qsem0, qsem1)
        sblos = (sblo0, sblo1)
        sbhis = (sbhi0, sbhi1)
        iblos = (iblo0, iblo1)
        ibhis = (ibhi0, ibhi1)
        ssems = (ssem0, ssem1)
        NCT = BPC * NCHUNK

        def q_src(j):
            b_local = j // NCHUNK
            chunk = j - b_local * NCHUNK
            b = c * BPC + b_local
            n0 = t * NT + chunk * CH
            return q_hbm.at[pl.ds((b * N + n0) * D, CH * D)]

        pltpu.async_copy(q_src(0), qbufs[0], qsems[0])

        def wait_scatter(p):
            pltpu.make_async_copy(sblos[p], table.at[iblos[p]],
                                  ssems[p]).wait()
            pltpu.make_async_copy(sbhis[p], table.at[ibhis[p]],
                                  ssems[p]).wait()

        def step(j, p):
            @pl.when(j < NCT - 1)
            def _():
                pltpu.async_copy(q_src(j + 1), qbufs[1 - p], qsems[1 - p])
            pltpu.make_async_copy(q_src(j), qbufs[p], qsems[p]).wait()

            @pl.when(j >= 2)
            def _():
                wait_scatter(p)
            b_local = j // NCHUNK
            chunk = j - b_local * NCHUNK
            base = b_local * NT * M + chunk * CHM
            for jj in range(CHM // L):
                iv = ibig[pl.ds(base + jj * L, L)]
                iblos[p][pl.ds(jj * L, L)] = iv
                ibhis[p][pl.ds(jj * L, L)] = iv + BPC * K
            qb = qbufs[p]
            slo = sblos[p]
            shi = sbhis[p]

            def build(i, carry3):
                wrow = wbig[pl.ds(base + i * M, L)]
                qv = [qb[pl.ds(i * D + v * L, L)] for v in range(NVR)]
                for s in range(M):
                    r = i * M + s
                    wv = jnp.full((L,), wrow[s], jnp.float32)
                    for v in range(NVR):
                        dst = slo if v < 8 else shi
                        dst[r, pl.ds((v % 8) * L, L)] = qv[v] * wv
                return carry3
            lax.fori_loop(0, CH, build, 0)
            pltpu.async_copy(slo, table.at[iblos[p]], ssems[p], add=True)
            pltpu.async_copy(shi, table.at[ibhis[p]], ssems[p], add=True)

        def super_step(g, carry):
            step(2 * g, 0)
            step(2 * g + 1, 1)
            return carry
        lax.fori_loop(0, NCT // 2, super_step, 0)
        wait_scatter(0)
        wait_scatter(1)
        plsc.subcore_barrier()

        RPT = BPC * K // NS
        r0 = t * RPT
        d_lo = pltpu.async_copy(
            table.at[pl.ds(r0, RPT)],
            praw_hbm.at[pl.ds(c * BPC * K + r0, RPT), pl.ds(0, HW)], lsem)
        d_hi = pltpu.async_copy(
            table.at[pl.ds(BPC * K + r0, RPT)],
            praw_hbm.at[pl.ds(c * BPC * K + r0, RPT), pl.ds(HW, HW)], lsem)
        d_lo.wait()
        d_hi.wait()

    return sc_scatter


def _make_sc_gather():
    @functools.partial(
        pl.kernel,
        out_type=[jax.ShapeDtypeStruct((B * N * D,), jnp.float32)],
        mesh=_sc_mesh(),
        scratch_types=[
            pltpu.VMEM((CHM, D), jnp.float32),
            pltpu.VMEM((CHM, D), jnp.float32),
            pltpu.VMEM((CHM,), jnp.int32),
            pltpu.VMEM((CHM,), jnp.int32),
            pltpu.VMEM((BPC * NT * M,), jnp.int32),
            pltpu.VMEM((BPC * NT * M + L,), jnp.float32),
            pltpu.VMEM((CH * D,), jnp.float32),
            pltpu.SemaphoreType.DMA,
            pltpu.SemaphoreType.DMA,
            pltpu.SemaphoreType.DMA,
        ],
    )
    def sc_gather(pn_hbm, isc_hbm, wg_hbm, g_hbm,
                  gbuf0, gbuf1, ibuf0, ibuf1, ibig, wbig, obuf,
                  lsem, gsem0, gsem1):
        c = lax.axis_index("c")
        t = lax.axis_index("s")
        coff = c * BPC * K

        pds = []
        for bl in range(BPC):
            b0 = c * BPC + bl
            pds.append(pltpu.async_copy(
                isc_hbm.at[pl.ds(b0 * NM + t * NT * M, NT * M)],
                ibig.at[pl.ds(bl * NT * M, NT * M)], lsem))
            pds.append(pltpu.async_copy(
                wg_hbm.at[pl.ds(b0 * NM + t * NT * M, NT * M)],
                wbig.at[pl.ds(bl * NT * M, NT * M)], lsem))
        for dp in pds:
            dp.wait()

        gbufs = (gbuf0, gbuf1)
        ibufs = (ibuf0, ibuf1)
        gsems = (gsem0, gsem1)
        NCT = BPC * NCHUNK

        def fill_idx(j, p):
            b_local = j // NCHUNK
            chunk = j - b_local * NCHUNK
            base = b_local * NT * M + chunk * CHM
            for jj in range(CHM // L):
                ibufs[p][pl.ds(jj * L, L)] = (
                    ibig[pl.ds(base + jj * L, L)] + coff)

        fill_idx(0, 0)
        pltpu.async_copy(pn_hbm.at[ibuf0], gbuf0, gsem0)

        def step(j, p):
            @pl.when(j < NCT - 1)
            def _():
                fill_idx(j + 1, 1 - p)
                pltpu.async_copy(pn_hbm.at[ibufs[1 - p]], gbufs[1 - p],
                                 gsems[1 - p])
            pltpu.make_async_copy(pn_hbm.at[ibufs[p]], gbufs[p],
                                  gsems[p]).wait()
            b_local = j // NCHUNK
            chunk = j - b_local * NCHUNK
            base = b_local * NT * M + chunk * CHM
            b = c * BPC + b_local
            n0 = t * NT + chunk * CH
            gb = gbufs[p]

            def wreduce(i, carry3):
                wrow = wbig[pl.ds(base + i * M, L)]
                accs = [jnp.zeros((L,), jnp.float32) for _ in range(NVR)]
                for s in range(M):
                    r = i * M + s
                    wv = jnp.full((L,), wrow[s], jnp.float32)
                    for v in range(NVR):
                        accs[v] = accs[v] + gb[r, pl.ds(v * L, L)] * wv
                for v in range(NVR):
                    obuf[pl.ds(i * D + v * L, L)] = accs[v]
                return carry3
            lax.fori_loop(0, CH, wreduce, 0)
            pltpu.sync_copy(obuf, g_hbm.at[pl.ds((b * N + n0) * D, CH * D)])

        def super_step(g, carry):
            step(2 * g, 0)
            step(2 * g + 1, 1)
            return carry
        lax.fori_loop(0, NCT // 2, super_step, 0)

    return sc_gather


_SC_SCATTER = _make_sc_scatter()
_SC_GATHER = _make_sc_gather()

_KCH = 256


_NSC = 512


def _tc_wsum_body(il_ref, ih_ref, w_ref, o_ref):
    kv = lax.broadcasted_iota(jnp.int32, (_KCH, 1), 0)
    for b in range(B):
        acc = jnp.zeros((_KCH, K // _KCH), jnp.float32)
        for nc in range(NM // _NSC):
            ilrow = il_ref[b:b + 1, nc * _NSC:(nc + 1) * _NSC]
            ihrow = ih_ref[b:b + 1, nc * _NSC:(nc + 1) * _NSC]
            wrow = w_ref[b:b + 1, nc * _NSC:(nc + 1) * _NSC]
            eq = jnp.where(kv == ilrow, 1.0, 0.0)
            wjs = jnp.concatenate(
                [jnp.where(ihrow == j, wrow, 0.0)
                 for j in range(K // _KCH)], axis=0)
            acc = acc + lax.dot_general(
                eq, wjs, (((1,), (1,)), ((), ())),
                preferred_element_type=jnp.float32)
        o_ref[b, :, :] = acc.T


def _tc_wsum(idxl, idxh, wp):
    return pl.pallas_call(
        _tc_wsum_body,
        grid=(1,),
        in_specs=[
            pl.BlockSpec((B, NM), lambda i: (0, 0)),
            pl.BlockSpec((B, NM), lambda i: (0, 0)),
            pl.BlockSpec((B, NM), lambda i: (0, 0)),
        ],
        out_specs=pl.BlockSpec((B, K // _KCH, _KCH), lambda i: (0, 0, 0)),
        out_shape=jax.ShapeDtypeStruct((B, K // _KCH, _KCH), jnp.float32),
    )(idxl, idxh, wp)


def _tc_norm_body(x_ref, ws_ref, w_ref, b_ref, on_ref, oo_ref):
    pn = x_ref[...] / (ws_ref[...][:, 0:1] + 1e-6)
    on_ref[...] = pn
    oo_ref[...] = jnp.dot(pn, w_ref[...],
                          preferred_element_type=jnp.float32) + b_ref[...]


def _tc_norm(praw, wsumb, wppT, bpp):
    R = B * K // 8
    return pl.pallas_call(
        _tc_norm_body,
        grid=(8,),
        in_specs=[
            pl.BlockSpec((R, D), lambda i: (i, 0)),
            pl.BlockSpec((R, 128), lambda i: (i, 0)),
            pl.BlockSpec((D, D), lambda i: (0, 0)),
            pl.BlockSpec((1, D), lambda i: (0, 0)),
        ],
        out_specs=[
            pl.BlockSpec((R, D), lambda i: (i, 0)),
            pl.BlockSpec((R, D), lambda i: (i, 0)),
        ],
        out_shape=[
            jax.ShapeDtypeStruct((B * K, D), jnp.float32),
            jax.ShapeDtypeStruct((B * K, D), jnp.float32),
        ],
    )(praw, wsumb, wppT, bpp)


def _tc_update_body(q_ref, g_ref, sw_ref, wpp_ref, bpp_ref, wup_ref, bup_ref,
                    lng_ref, lnb_ref, o_ref):
    Qc = D // 4
    g = jnp.dot(g_ref[...], wpp_ref[...], preferred_element_type=jnp.float32)
    g = g + sw_ref[...][:, 0:1] * bpp_ref[...]
    qb = q_ref[...]
    pr, pi_, pj, pk = (qb[:, :Qc], qb[:, Qc:2 * Qc],
                       qb[:, 2 * Qc:3 * Qc], qb[:, 3 * Qc:])
    xr, xi, xj, xk = (g[:, :Qc], g[:, Qc:2 * Qc],
                      g[:, 2 * Qc:3 * Qc], g[:, 3 * Qc:])
    hr = pr * xr - pi_ * xi - pj * xj - pk * xk
    hi = pr * xi + pi_ * xr + pj * xk - pk * xj
    hj = pr * xj - pi_ * xk + pj * xr + pk * xi
    hk = pr * xk + pi_ * xj - pj * xi + pk * xr
    msg = jnp.concatenate([hr, hi, hj, hk], axis=1)
    out = jnp.dot(msg, wup_ref[...],
                  preferred_element_type=jnp.float32) + bup_ref[...]
    x = qb + out
    parts = []
    lng = lng_ref[...]
    lnb = lnb_ref[...]
    for ci in range(4):
        xc = x[:, ci * Qc:(ci + 1) * Qc]
        mu = jnp.mean(xc, axis=1, keepdims=True)
        xm = xc - mu
        var = jnp.mean(xm * xm, axis=1, keepdims=True)
        y = xm * lax.rsqrt(var + 1e-5)
        parts.append(y * lng[:, ci * Qc:(ci + 1) * Qc]
                     + lnb[:, ci * Qc:(ci + 1) * Qc])
    o_ref[...] = jnp.concatenate(parts, axis=1)


def _tc_update(qf, gf, swf, wppT, bpp, wupT, bup, lng, lnb):
    R = B * N // 9
    return pl.pallas_call(
        _tc_update_body,
        grid=(9,),
        in_specs=[
            pl.BlockSpec((R, D), lambda i: (i, 0)),
            pl.BlockSpec((R, D), lambda i: (i, 0)),
            pl.BlockSpec((R, 128), lambda i: (i, 0)),
            pl.BlockSpec((D, D), lambda i: (0, 0)),
            pl.BlockSpec((1, D), lambda i: (0, 0)),
            pl.BlockSpec((D, D), lambda i: (0, 0)),
            pl.BlockSpec((1, D), lambda i: (0, 0)),
            pl.BlockSpec((1, D), lambda i: (0, 0)),
            pl.BlockSpec((1, D), lambda i: (0, 0)),
        ],
        out_specs=pl.BlockSpec((R, D), lambda i: (i, 0)),
        out_shape=jax.ShapeDtypeStruct((B * N, D), jnp.float32),
    )(qf, gf, swf, wppT, bpp, wupT, bup, lng, lnb)


def _quat_weight(r, i, j, k):
    return jnp.concatenate([
        jnp.concatenate([r, -i, -j, -k], 1),
        jnp.concatenate([i, r, -k, j], 1),
        jnp.concatenate([j, k, r, -i], 1),
        jnp.concatenate([k, -j, i, r], 1)], 0)


def kernel(q, assign_idx, assign_w, contribute_mask,
           pp_r, pp_i, pp_j, pp_k, pp_b,
           up_r, up_i, up_j, up_k, up_b,
           ln_gr, ln_br, ln_gi, ln_bi, ln_gj, ln_bj, ln_gk, ln_bk):
    agg_w = assign_w * contribute_mask[..., None]
    idx = assign_idx.astype(jnp.int32)
    core_off = (jnp.arange(B, dtype=jnp.int32) % BPC) * K
    isc = (idx + core_off[:, None, None]).reshape(B * NM)
    ws = agg_w.reshape(B * NM)
    wg = assign_w.reshape(B * NM)

    praw = _SC_SCATTER(q.reshape(B * N * D), isc, ws)[0]

    idxr = idx.reshape(B, NM)
    wsum = _tc_wsum(idxr & 255, idxr >> 8, agg_w.reshape(B, NM))
    wsumb = jnp.broadcast_to(wsum.reshape(B * K, 1), (B * K, 128))

    wppT = _quat_weight(pp_r, pp_i, pp_j, pp_k).T
    wupT = _quat_weight(up_r, up_i, up_j, up_k).T
    proto_norm, proto_out = _tc_norm(praw, wsumb, wppT, pp_b.reshape(1, D))

    g_raw = _SC_GATHER(proto_norm, isc, wg)[0]

    sw = jnp.broadcast_to(assign_w.sum(-1).reshape(B * N, 1), (B * N, 128))
    lng = jnp.concatenate([ln_gr, ln_gi, ln_gj, ln_gk]).reshape(1, D)
    lnb = jnp.concatenate([ln_br, ln_bi, ln_bj, ln_bk]).reshape(1, D)
    qn = _tc_update(q.reshape(B * N, D), g_raw.reshape(B * N, D), sw,
                    wppT, pp_b.reshape(1, D), wupT, up_b.reshape(1, D),
                    lng, lnb)
    return qn.reshape(B, N, D), proto_out.reshape(B, K, D)
